# Initial kernel scaffold; baseline (speedup 1.0000x reference)
import jax
import jax.numpy as jnp
from jax.experimental import pallas as pl


def kernel(x, edge_index, tg_mask, batch, ptr, feature_fc_w, feature_fc_b, edge_fc_w, edge_fc_b, gat_w, gat_b, gat_out_w, gat_out_b, inv_w, inv_b, mix_w, mix_b):
    raise NotImplementedError("write your pallas kernel here")


# speedup vs baseline: 28.1763x; 28.1763x over previous
import functools

import jax
import jax.numpy as jnp
import numpy as np
from jax import lax
from jax.experimental import pallas as pl
from jax.experimental.pallas import tpu as pltpu
from jax.experimental.pallas import tpu_sc as plsc

N = 10000
E = 320000
DIM_IN = 128

NS = 16
NP = 10240
NPT = NP // NS
NP4 = NP * 4
EPW = E // NS
C = 2000
L = 16
NR = NP4 // 128
GR = 80
RPT = NR // NS


def _raw_consts():
    nkey = jax.random.key(42)

    def logit_noise(key, shape):
        u = jax.random.uniform(key, shape, minval=1e-10, maxval=1.0 - 1e-10,
                               dtype=jnp.float32)
        return jnp.log(u) - jnp.log(1.0 - u)

    n0 = logit_noise(jax.random.fold_in(nkey, 0), (N, DIM_IN - 2))
    n1 = logit_noise(jax.random.fold_in(nkey, 1), (E, 1))
    perm = jax.random.permutation(jax.random.fold_in(nkey, 2), N)
    return n0, n1, perm


def _consts_np():
    cpu = jax.devices("cpu")[0]
    with jax.default_device(cpu):
        n0, n1, perm = _raw_consts()
        n0, n1, perm = np.asarray(n0), np.asarray(n1), np.asarray(perm)
    n0_pad = np.zeros((NP, DIM_IN), dtype=np.float32)
    n0_pad[:N, : DIM_IN - 2] = n0
    perm_pad = np.zeros((NP,), dtype=np.int32)
    perm_pad[:N] = perm.astype(np.int32)
    return n0_pad, n1.reshape(E).astype(np.float32), perm_pad


try:
    _CONSTS = _consts_np()
except Exception:
    _CONSTS = None


def _get_consts():
    if _CONSTS is not None:
        return _CONSTS
    n0, n1, perm = _raw_consts()
    n0_pad = jnp.zeros((NP, DIM_IN), jnp.float32).at[:N, : DIM_IN - 2].set(n0)
    perm_pad = jnp.zeros((NP,), jnp.int32).at[:N].set(perm.astype(jnp.int32))
    return n0_pad, n1.reshape(E), perm_pad

_BN = 2048


def _tc_body(x_ref, nz_ref, wf_ref, bf_ref, b1_ref, b2_ref, c0_ref,
             mask_ref, tab_ref):
    xb = x_ref[...]
    ml = jnp.dot(xb, wf_ref[...], preferred_element_type=jnp.float32)
    mask = jax.nn.sigmoid(ml + bf_ref[...] + nz_ref[...])
    mask_ref[...] = mask
    col = lax.broadcasted_iota(jnp.int32, xb.shape, 1)
    y = jnp.where(col < DIM_IN - 2, xb * mask, xb)
    tab = (jnp.dot(y, b1_ref[...], preferred_element_type=jnp.float32)
           + jnp.dot(xb, b2_ref[...], preferred_element_type=jnp.float32)
           + c0_ref[...])
    tab_ref[...] = tab


def _tc_dense(x_pad, nz_pad, wf_pad, bf_row, b1, b2, c0_row):
    grid = (NP // _BN,)
    return pl.pallas_call(
        _tc_body,
        grid=grid,
        in_specs=[
            pl.BlockSpec((_BN, DIM_IN), lambda i: (i, 0)),
            pl.BlockSpec((_BN, DIM_IN), lambda i: (i, 0)),
            pl.BlockSpec((DIM_IN, DIM_IN), lambda i: (0, 0)),
            pl.BlockSpec((1, DIM_IN), lambda i: (0, 0)),
            pl.BlockSpec((DIM_IN, 8), lambda i: (0, 0)),
            pl.BlockSpec((DIM_IN, 8), lambda i: (0, 0)),
            pl.BlockSpec((1, 8), lambda i: (0, 0)),
        ],
        out_specs=[
            pl.BlockSpec((_BN, DIM_IN), lambda i: (i, 0)),
            pl.BlockSpec((_BN, 8), lambda i: (i, 0)),
        ],
        out_shape=[
            jax.ShapeDtypeStruct((NP, DIM_IN), jnp.float32),
            jax.ShapeDtypeStruct((NP, 8), jnp.float32),
        ],
    )(x_pad, nz_pad, wf_pad, bf_row, b1, b2, c0_row)


def _sc_body(tab_ref, perm_ref, src_ref, dst_ref, nzE_ref, init0_ref, initz_ref,
             acc_out, att_out,
             t0, t1, t2, t3, t4, t5, acc, srcb, dstb, nzb, attb, permb,
             tmpacc, sh_up0, sh_up1, sh_acc):
    wid = lax.axis_index("s")
    nbase = pl.multiple_of(wid * NPT, 8)

    pltpu.sync_copy(tab_ref.at[pl.ds(2 * NP, NP)], t0)
    pltpu.sync_copy(tab_ref.at[pl.ds(3 * NP, NP)], t1)
    pltpu.sync_copy(tab_ref.at[pl.ds(4 * NP, NP)], t2)
    pltpu.sync_copy(tab_ref.at[pl.ds(5 * NP, NP)], t3)
    pltpu.sync_copy(tab_ref.at[pl.ds(6 * NP, NP)], t4)
    pltpu.sync_copy(tab_ref.at[pl.ds(7 * NP, NP)], t5)
    @pl.when(wid == 0)
    def _():
        pltpu.sync_copy(init0_ref, acc)

    @pl.when(wid != 0)
    def _():
        pltpu.sync_copy(initz_ref, acc)

    pltpu.sync_copy(perm_ref.at[pl.ds(nbase, NPT)], permb)

    def pa(j, carry):
        p = permb[pl.ds(j * L, L)]
        g0 = plsc.load_gather(t2, [p])
        g1 = plsc.load_gather(t3, [p])
        o = nbase + j * L
        t0[pl.ds(o, L)] = t0[pl.ds(o, L)] + g0
        t1[pl.ds(o, L)] = t1[pl.ds(o, L)] + g1
        return carry

    lax.fori_loop(0, NPT // L, pa, 0)
    pltpu.sync_copy(t0.at[pl.ds(nbase, NPT)], sh_up0.at[pl.ds(nbase, NPT)])
    pltpu.sync_copy(t1.at[pl.ds(nbase, NPT)], sh_up1.at[pl.ds(nbase, NPT)])
    pltpu.sync_copy(tab_ref.at[pl.ds(0, NP)], t2)
    pltpu.sync_copy(tab_ref.at[pl.ds(NP, NP)], t3)
    plsc.subcore_barrier()
    pltpu.sync_copy(sh_up0, t0)
    pltpu.sync_copy(sh_up1, t1)

    for ch in range(EPW // C):
        eb = pl.multiple_of(wid * EPW + ch * C, 8)
        pltpu.sync_copy(src_ref.at[pl.ds(eb, C)], srcb)
        pltpu.sync_copy(dst_ref.at[pl.ds(eb, C)], dstb)
        pltpu.sync_copy(nzE_ref.at[pl.ds(eb, C)], nzb)

        def ej(j, carry):
            s = srcb[pl.ds(j * L, L)]
            d = dstb[pl.ds(j * L, L)]
            esv = plsc.load_gather(t4, [s])
            edv = plsc.load_gather(t5, [d])
            lg = esv + edv + nzb[pl.ds(j * L, L)]
            att = 1.0 / (1.0 + jnp.exp(-lg))
            attb[pl.ds(j * L, L)] = att
            uc0 = plsc.load_gather(t2, [s])
            uc1 = plsc.load_gather(t3, [s])
            up0 = plsc.load_gather(t0, [s])
            up1 = plsc.load_gather(t1, [s])
            r = lax.shift_right_logical(d, 5)
            lb = lax.shift_left(d & 31, 2)
            plsc.addupdate_scatter(acc, [r, lb], att * uc0)
            plsc.addupdate_scatter(acc, [r, lb + 1], att * uc1)
            plsc.addupdate_scatter(acc, [r, lb + 2], att * up0)
            plsc.addupdate_scatter(acc, [r, lb + 3], att * up1)
            return carry

        lax.fori_loop(0, C // L, ej, 0)
        pltpu.sync_copy(attb, att_out.at[pl.ds(eb, C)])

    iota16 = lax.iota(jnp.int32, L)

    @pl.when(wid == 0)
    def _():
        pltpu.sync_copy(acc, sh_acc)

    plsc.subcore_barrier()

    @pl.when(wid != 0)
    def _():
        for g in range(NR // L):
            pltpu.sync_copy(acc.at[pl.ds(L * g, L)],
                            sh_acc.at[iota16 + L * g], add=True)

    plsc.subcore_barrier()

    @pl.when(wid < 8)
    def _():
        ob = pl.multiple_of(wid * 40, 8)
        pltpu.sync_copy(sh_acc.at[pl.ds(ob, 40)], tmpacc)
        pltpu.sync_copy(tmpacc, acc_out.at[pl.ds(ob, 40)])


def _sc_edges(table_t, perm_pad, src, dst, nz, init0, initz):
    mesh = plsc.VectorSubcoreMesh(core_axis_name="c", subcore_axis_name="s",
                                  num_cores=1)
    f32 = jnp.float32
    i32 = jnp.int32
    kfn = pl.kernel(
        _sc_body,
        compiler_params=pltpu.CompilerParams(needs_layout_passes=False),
        out_type=[
            jax.ShapeDtypeStruct((NR, 128), f32),
            jax.ShapeDtypeStruct((E,), f32),
        ],
        mesh=mesh,
        scratch_types=[
            pltpu.VMEM((NP,), f32),
            pltpu.VMEM((NP,), f32),
            pltpu.VMEM((NP,), f32),
            pltpu.VMEM((NP,), f32),
            pltpu.VMEM((NP,), f32),
            pltpu.VMEM((NP,), f32),
            pltpu.VMEM((NR, 128), f32),
            pltpu.VMEM((C,), i32),
            pltpu.VMEM((C,), i32),
            pltpu.VMEM((C,), f32),
            pltpu.VMEM((C,), f32),
            pltpu.VMEM((NPT,), i32),
            pltpu.VMEM((40, 128), f32),
            pltpu.VMEM_SHARED((NP,), f32),
            pltpu.VMEM_SHARED((NP,), f32),
            pltpu.VMEM_SHARED((NR, 128), f32),
        ],
    )
    return kfn(table_t, perm_pad, src, dst, nz, init0, initz)


def kernel(x, edge_index, tg_mask, batch, ptr,
           feature_fc_w, feature_fc_b, edge_fc_w, edge_fc_b,
           gat_w, gat_b, gat_out_w, gat_out_b,
           inv_w, inv_b, mix_w, mix_b):
    n0_pad, n1, perm_pad = _get_consts()
    f32 = jnp.float32

    A = gat_w @ gat_out_w
    wc = A @ inv_w
    wp = A @ mix_w
    gba = gat_b @ gat_out_w
    bc = gba @ inv_w
    bp = gba @ mix_w
    wp_z = wp.at[DIM_IN - 2:].set(0.0)
    z2 = jnp.zeros((DIM_IN, 2), f32)
    b1 = jnp.concatenate([wc, wp, -wp_z, z2], axis=1)
    b2 = jnp.concatenate([jnp.zeros((DIM_IN, 4), f32), wp_z,
                          edge_fc_w[:DIM_IN], edge_fc_w[DIM_IN:]], axis=1)
    c0 = jnp.concatenate([bc, bp, jnp.zeros((2,), f32), edge_fc_b,
                          jnp.zeros((1,), f32)])
    ccc = gat_out_b @ inv_w + inv_b
    ccp = gat_out_b @ mix_w + mix_b
    cv16 = jnp.tile(jnp.concatenate([ccc, ccp]), 4)

    wf_pad = jnp.zeros((DIM_IN, DIM_IN), f32).at[:, : DIM_IN - 2].set(feature_fc_w)
    bf_row = jnp.zeros((1, DIM_IN), f32).at[0, : DIM_IN - 2].set(feature_fc_b)

    x_pad = jnp.pad(x, ((0, NP - N), (0, 0)))
    mask_pad, tab = _tc_dense(x_pad, jnp.asarray(n0_pad), wf_pad, bf_row,
                              b1, b2, c0.reshape(1, 8))
    table_t = tab.T.reshape(-1)

    init0 = jnp.tile(cv16.reshape(1, L), (NR, 8))
    initz = jnp.zeros((NR, 128), f32)
    acc2d, att = _sc_edges(table_t, jnp.asarray(perm_pad),
                           edge_index[0], edge_index[1], jnp.asarray(n1),
                           init0, initz)
    accr = acc2d.reshape(NP, 4)[:N]
    perm_pred = accr[:, 2:4]
    xc_pred = accr[:, 0:2]
    feature_mask = mask_pad[:N, : DIM_IN - 2]
    edge_att = att.reshape(E, 1)
    return (perm_pred, xc_pred, feature_mask, edge_att)


# speedup vs baseline: 29.5169x; 1.0476x over previous
import functools

import jax
import jax.numpy as jnp
import numpy as np
from jax import lax
from jax.experimental import pallas as pl
from jax.experimental.pallas import tpu as pltpu
from jax.experimental.pallas import tpu_sc as plsc

N = 10000
E = 320000
DIM_IN = 128

NS = 16
NP = 10240
NPT = NP // NS
NP4 = NP * 4
EPW = E // NS
C = 2000
L = 16
NR = NP4 // 128
GR = 80
RPT = NR // NS


def _raw_consts():
    nkey = jax.random.key(42)

    def logit_noise(key, shape):
        u = jax.random.uniform(key, shape, minval=1e-10, maxval=1.0 - 1e-10,
                               dtype=jnp.float32)
        return jnp.log(u) - jnp.log(1.0 - u)

    n0 = logit_noise(jax.random.fold_in(nkey, 0), (N, DIM_IN - 2))
    n1 = logit_noise(jax.random.fold_in(nkey, 1), (E, 1))
    perm = jax.random.permutation(jax.random.fold_in(nkey, 2), N)
    return n0, n1, perm


def _consts_np():
    cpu = jax.devices("cpu")[0]
    with jax.default_device(cpu):
        n0, n1, perm = _raw_consts()
        n0, n1, perm = np.asarray(n0), np.asarray(n1), np.asarray(perm)
    n0_pad = np.zeros((NP, DIM_IN), dtype=np.float32)
    n0_pad[:N, : DIM_IN - 2] = n0
    perm_pad = np.zeros((NP,), dtype=np.int32)
    perm_pad[:N] = perm.astype(np.int32)
    return n0_pad, n1.reshape(E).astype(np.float32), perm_pad


try:
    _CONSTS = _consts_np()
except Exception:
    _CONSTS = None


def _get_consts():
    if _CONSTS is not None:
        return _CONSTS
    n0, n1, perm = _raw_consts()
    n0_pad = jnp.zeros((NP, DIM_IN), jnp.float32).at[:N, : DIM_IN - 2].set(n0)
    perm_pad = jnp.zeros((NP,), jnp.int32).at[:N].set(perm.astype(jnp.int32))
    return n0_pad, n1.reshape(E), perm_pad

_BN = 2048


def _tc_body(x_ref, nz_ref, wf_ref, bf_ref, b1_ref, b2_ref, c0_ref,
             mask_ref, tab_ref):
    xb = x_ref[...]
    ml = jnp.dot(xb, wf_ref[...], preferred_element_type=jnp.float32)
    mask = jax.nn.sigmoid(ml + bf_ref[...] + nz_ref[...])
    mask_ref[...] = mask[:, : DIM_IN - 2]
    col = lax.broadcasted_iota(jnp.int32, xb.shape, 1)
    y = jnp.where(col < DIM_IN - 2, xb * mask, xb)
    tab = (jnp.dot(y, b1_ref[...], preferred_element_type=jnp.float32)
           + jnp.dot(xb, b2_ref[...], preferred_element_type=jnp.float32)
           + c0_ref[...])
    tab_ref[...] = tab.T


def _tc_dense(x, nz_pad, wf_pad, bf_row, b1, b2, c0_row):
    grid = (NP // _BN,)
    return pl.pallas_call(
        _tc_body,
        grid=grid,
        in_specs=[
            pl.BlockSpec((_BN, DIM_IN), lambda i: (i, 0)),
            pl.BlockSpec((_BN, DIM_IN), lambda i: (i, 0)),
            pl.BlockSpec((DIM_IN, DIM_IN), lambda i: (0, 0)),
            pl.BlockSpec((1, DIM_IN), lambda i: (0, 0)),
            pl.BlockSpec((DIM_IN, 8), lambda i: (0, 0)),
            pl.BlockSpec((DIM_IN, 8), lambda i: (0, 0)),
            pl.BlockSpec((1, 8), lambda i: (0, 0)),
        ],
        out_specs=[
            pl.BlockSpec((_BN, DIM_IN - 2), lambda i: (i, 0)),
            pl.BlockSpec((8, _BN), lambda i: (0, i)),
        ],
        out_shape=[
            jax.ShapeDtypeStruct((N, DIM_IN - 2), jnp.float32),
            jax.ShapeDtypeStruct((8, NP), jnp.float32),
        ],
    )(x, nz_pad, wf_pad, bf_row, b1, b2, c0_row)


def _sc_body(tab_ref, perm_ref, src_ref, dst_ref, nzE_ref, init0_ref, initz_ref,
             acc_out, att_out,
             t0, t1, t2, t3, t4, t5, acc, srcb, dstb, nzb, attb, permb,
             tmpacc, sh_up0, sh_up1, sh_acc):
    wid = lax.axis_index("s")
    nbase = pl.multiple_of(wid * NPT, 8)

    pltpu.sync_copy(tab_ref.at[pl.ds(2 * NP, NP)], t0)
    pltpu.sync_copy(tab_ref.at[pl.ds(3 * NP, NP)], t1)
    pltpu.sync_copy(tab_ref.at[pl.ds(4 * NP, NP)], t2)
    pltpu.sync_copy(tab_ref.at[pl.ds(5 * NP, NP)], t3)
    pltpu.sync_copy(tab_ref.at[pl.ds(6 * NP, NP)], t4)
    pltpu.sync_copy(tab_ref.at[pl.ds(7 * NP, NP)], t5)
    @pl.when(wid == 0)
    def _():
        pltpu.sync_copy(init0_ref, acc)

    @pl.when(wid != 0)
    def _():
        pltpu.sync_copy(initz_ref, acc)

    pltpu.sync_copy(perm_ref.at[pl.ds(nbase, NPT)], permb)

    def pa(j, carry):
        p = permb[pl.ds(j * L, L)]
        g0 = plsc.load_gather(t2, [p])
        g1 = plsc.load_gather(t3, [p])
        o = nbase + j * L
        t0[pl.ds(o, L)] = t0[pl.ds(o, L)] + g0
        t1[pl.ds(o, L)] = t1[pl.ds(o, L)] + g1
        return carry

    lax.fori_loop(0, NPT // L, pa, 0)
    pltpu.sync_copy(t0.at[pl.ds(nbase, NPT)], sh_up0.at[pl.ds(nbase, NPT)])
    pltpu.sync_copy(t1.at[pl.ds(nbase, NPT)], sh_up1.at[pl.ds(nbase, NPT)])
    pltpu.sync_copy(tab_ref.at[pl.ds(0, NP)], t2)
    pltpu.sync_copy(tab_ref.at[pl.ds(NP, NP)], t3)
    plsc.subcore_barrier()
    pltpu.sync_copy(sh_up0, t0)
    pltpu.sync_copy(sh_up1, t1)

    for ch in range(EPW // C):
        eb = pl.multiple_of(wid * EPW + ch * C, 8)
        pltpu.sync_copy(src_ref.at[pl.ds(eb, C)], srcb)
        pltpu.sync_copy(dst_ref.at[pl.ds(eb, C)], dstb)
        pltpu.sync_copy(nzE_ref.at[pl.ds(eb, C)], nzb)

        def ej(j, carry):
            s = srcb[pl.ds(j * L, L)]
            d = dstb[pl.ds(j * L, L)]
            esv = plsc.load_gather(t4, [s])
            edv = plsc.load_gather(t5, [d])
            lg = esv + edv + nzb[pl.ds(j * L, L)]
            att = 1.0 / (1.0 + jnp.exp(-lg))
            attb[pl.ds(j * L, L)] = att
            uc0 = plsc.load_gather(t2, [s])
            uc1 = plsc.load_gather(t3, [s])
            up0 = plsc.load_gather(t0, [s])
            up1 = plsc.load_gather(t1, [s])
            r = lax.shift_right_logical(d, 5)
            lb = lax.shift_left(d & 31, 2)
            plsc.addupdate_scatter(acc, [r, lb], att * uc0)
            plsc.addupdate_scatter(acc, [r, lb + 1], att * uc1)
            plsc.addupdate_scatter(acc, [r, lb + 2], att * up0)
            plsc.addupdate_scatter(acc, [r, lb + 3], att * up1)
            return carry

        lax.fori_loop(0, C // L, ej, 0)
        pltpu.sync_copy(attb, att_out.at[pl.ds(eb, C)])

    iota16 = lax.iota(jnp.int32, L)

    @pl.when(wid == 0)
    def _():
        pltpu.sync_copy(acc, sh_acc)

    plsc.subcore_barrier()

    @pl.when(wid != 0)
    def _():
        for g in range(NR // L):
            pltpu.sync_copy(acc.at[pl.ds(L * g, L)],
                            sh_acc.at[iota16 + L * g], add=True)

    plsc.subcore_barrier()

    @pl.when(wid < 8)
    def _():
        ob = pl.multiple_of(wid * 40, 8)
        pltpu.sync_copy(sh_acc.at[pl.ds(ob, 40)], tmpacc)
        pltpu.sync_copy(tmpacc, acc_out.at[pl.ds(ob, 40)])


def _sc_edges(table_t, perm_pad, src, dst, nz, init0, initz):
    mesh = plsc.VectorSubcoreMesh(core_axis_name="c", subcore_axis_name="s",
                                  num_cores=1)
    f32 = jnp.float32
    i32 = jnp.int32
    kfn = pl.kernel(
        _sc_body,
        compiler_params=pltpu.CompilerParams(needs_layout_passes=False),
        out_type=[
            jax.ShapeDtypeStruct((NR, 128), f32),
            jax.ShapeDtypeStruct((E,), f32),
        ],
        mesh=mesh,
        scratch_types=[
            pltpu.VMEM((NP,), f32),
            pltpu.VMEM((NP,), f32),
            pltpu.VMEM((NP,), f32),
            pltpu.VMEM((NP,), f32),
            pltpu.VMEM((NP,), f32),
            pltpu.VMEM((NP,), f32),
            pltpu.VMEM((NR, 128), f32),
            pltpu.VMEM((C,), i32),
            pltpu.VMEM((C,), i32),
            pltpu.VMEM((C,), f32),
            pltpu.VMEM((C,), f32),
            pltpu.VMEM((NPT,), i32),
            pltpu.VMEM((40, 128), f32),
            pltpu.VMEM_SHARED((NP,), f32),
            pltpu.VMEM_SHARED((NP,), f32),
            pltpu.VMEM_SHARED((NR, 128), f32),
        ],
    )
    return kfn(table_t, perm_pad, src, dst, nz, init0, initz)


def kernel(x, edge_index, tg_mask, batch, ptr,
           feature_fc_w, feature_fc_b, edge_fc_w, edge_fc_b,
           gat_w, gat_b, gat_out_w, gat_out_b,
           inv_w, inv_b, mix_w, mix_b):
    n0_pad, n1, perm_pad = _get_consts()
    f32 = jnp.float32

    A = gat_w @ gat_out_w
    wc = A @ inv_w
    wp = A @ mix_w
    gba = gat_b @ gat_out_w
    bc = gba @ inv_w
    bp = gba @ mix_w
    wp_z = wp.at[DIM_IN - 2:].set(0.0)
    z2 = jnp.zeros((DIM_IN, 2), f32)
    b1 = jnp.concatenate([wc, wp, -wp_z, z2], axis=1)
    b2 = jnp.concatenate([jnp.zeros((DIM_IN, 4), f32), wp_z,
                          edge_fc_w[:DIM_IN], edge_fc_w[DIM_IN:]], axis=1)
    c0 = jnp.concatenate([bc, bp, jnp.zeros((2,), f32), edge_fc_b,
                          jnp.zeros((1,), f32)])
    ccc = gat_out_b @ inv_w + inv_b
    ccp = gat_out_b @ mix_w + mix_b
    cv16 = jnp.tile(jnp.concatenate([ccc, ccp]), 4)

    wf_pad = jnp.zeros((DIM_IN, DIM_IN), f32).at[:, : DIM_IN - 2].set(feature_fc_w)
    bf_row = jnp.zeros((1, DIM_IN), f32).at[0, : DIM_IN - 2].set(feature_fc_b)

    feature_mask, tab_t = _tc_dense(x, jnp.asarray(n0_pad), wf_pad, bf_row,
                                    b1, b2, c0.reshape(1, 8))
    table_t = tab_t.reshape(-1)

    init0 = jnp.tile(cv16.reshape(1, L), (NR, 8))
    initz = jnp.zeros((NR, 128), f32)
    acc2d, att = _sc_edges(table_t, jnp.asarray(perm_pad),
                           edge_index[0], edge_index[1], jnp.asarray(n1),
                           init0, initz)
    accr = acc2d.reshape(NP, 4)[:N]
    perm_pred = accr[:, 2:4]
    xc_pred = accr[:, 0:2]
    edge_att = att.reshape(E, 1)
    return (perm_pred, xc_pred, feature_mask, edge_att)


# speedup vs baseline: 38.1512x; 1.2925x over previous
import functools

import jax
import jax.numpy as jnp
import numpy as np
from jax import lax
from jax.experimental import pallas as pl
from jax.experimental.pallas import tpu as pltpu
from jax.experimental.pallas import tpu_sc as plsc

N = 10000
E = 320000
DIM_IN = 128

NS = 16
NP = 10240
NPT = NP // NS
NP4 = NP * 4
EPW = E // NS
C = 2000
L = 16
NR = NP4 // 128
GR = 80
RPT = NR // NS


def _raw_consts():
    nkey = jax.random.key(42)

    def logit_noise(key, shape):
        u = jax.random.uniform(key, shape, minval=1e-10, maxval=1.0 - 1e-10,
                               dtype=jnp.float32)
        return jnp.log(u) - jnp.log(1.0 - u)

    n0 = logit_noise(jax.random.fold_in(nkey, 0), (N, DIM_IN - 2))
    n1 = logit_noise(jax.random.fold_in(nkey, 1), (E, 1))
    perm = jax.random.permutation(jax.random.fold_in(nkey, 2), N)
    return n0, n1, perm


def _consts_np():
    cpu = jax.devices("cpu")[0]
    with jax.default_device(cpu):
        n0, n1, perm = _raw_consts()
        n0, n1, perm = np.asarray(n0), np.asarray(n1), np.asarray(perm)
    n0_pad = np.zeros((NP, DIM_IN - 2), dtype=np.float32)
    n0_pad[:N] = n0
    perm_pad = np.zeros((NP,), dtype=np.int32)
    perm_pad[:N] = perm.astype(np.int32)
    return n0_pad, n1.reshape(E).astype(np.float32), perm_pad


try:
    _CONSTS = _consts_np()
except Exception:
    _CONSTS = None


def _get_consts():
    if _CONSTS is not None:
        return _CONSTS
    n0, n1, perm = _raw_consts()
    n0_pad = jnp.zeros((NP, DIM_IN - 2), jnp.float32).at[:N].set(n0)
    perm_pad = jnp.zeros((NP,), jnp.int32).at[:N].set(perm.astype(jnp.int32))
    return n0_pad, n1.reshape(E), perm_pad

_BN = 2048


def _tc_body(x_ref, nz_ref, wf_ref, bf_ref, gw_ref, gb_ref, gow_ref,
             gob_ref, efw_ref, eb_ref, ivw_ref, ivb_ref, mxw_ref, mxb_ref,
             mask_ref, tab_ref, init_ref):
    f32 = jnp.float32
    xb = x_ref[...]
    ml = jnp.dot(xb, wf_ref[...], preferred_element_type=f32)
    mask = jax.nn.sigmoid(ml + bf_ref[...] + nz_ref[...])
    mask_ref[...] = mask
    x126 = xb[:, : DIM_IN - 2]
    xl = xb[:, DIM_IN - 2:]
    y = jnp.concatenate([x126 * mask, xl], axis=1)

    ivw = ivw_ref[...]
    mxw = mxw_ref[...]
    a = jnp.dot(gw_ref[...], gow_ref[...], preferred_element_type=f32)
    wc = jnp.dot(a, ivw, preferred_element_type=f32)
    wp = jnp.dot(a, mxw, preferred_element_type=f32)
    row = lax.broadcasted_iota(jnp.int32, wp.shape, 0)
    wp_z = jnp.where(row < DIM_IN - 2, wp, 0.0)
    gba = jnp.dot(gb_ref[...], gow_ref[...], preferred_element_type=f32)
    bc = jnp.dot(gba, ivw, preferred_element_type=f32)
    bp = jnp.dot(gba, mxw, preferred_element_type=f32)
    efw = efw_ref[...]
    ews = efw[: DIM_IN]
    ewd = efw[DIM_IN:]

    uc = jnp.dot(y, wc, preferred_element_type=f32) + bc
    upb = jnp.dot(y, wp, preferred_element_type=f32) + bp
    v = jnp.dot(xb - y, wp_z, preferred_element_type=f32)
    es = jnp.dot(xb, ews, preferred_element_type=f32) + eb_ref[...]
    ed = jnp.dot(xb, ewd, preferred_element_type=f32)
    tab = jnp.concatenate([uc, upb, v, es, ed], axis=1)
    tab_ref[...] = tab.T

    gob = gob_ref[...]
    ccc = jnp.dot(gob, ivw, preferred_element_type=f32) + ivb_ref[...]
    ccp = jnp.dot(gob, mxw, preferred_element_type=f32) + mxb_ref[...]
    cv4 = jnp.concatenate([ccc, ccp], axis=1)
    init_ref[...] = jnp.tile(cv4, (NR, 32))


def _tc_dense(x, nz_pad, feature_fc_w, feature_fc_b, gat_w, gat_b,
              gat_out_w, gat_out_b, edge_fc_w, edge_fc_b,
              inv_w, inv_b, mix_w, mix_b, interpret=False):
    grid = (NP // _BN,)
    d2 = DIM_IN - 2
    full = lambda shape: pl.BlockSpec(shape, lambda i: tuple(0 for _ in shape))
    return pl.pallas_call(
        _tc_body,
        grid=grid,
        in_specs=[
            pl.BlockSpec((_BN, DIM_IN), lambda i: (i, 0)),
            pl.BlockSpec((_BN, d2), lambda i: (i, 0)),
            full((DIM_IN, d2)),
            full((1, d2)),
            full((DIM_IN, DIM_IN)),
            full((1, DIM_IN)),
            full((DIM_IN, 2)),
            full((1, 2)),
            full((2 * DIM_IN, 1)),
            full((1, 1)),
            full((2, 2)),
            full((1, 2)),
            full((2, 2)),
            full((1, 2)),
        ],
        out_specs=[
            pl.BlockSpec((_BN, d2), lambda i: (i, 0)),
            pl.BlockSpec((8, _BN), lambda i: (0, i)),
            pl.BlockSpec((NR, 128), lambda i: (0, 0)),
        ],
        out_shape=[
            jax.ShapeDtypeStruct((N, d2), jnp.float32),
            jax.ShapeDtypeStruct((8, NP), jnp.float32),
            jax.ShapeDtypeStruct((NR, 128), jnp.float32),
        ],
        interpret=interpret,
    )(x, nz_pad, feature_fc_w, feature_fc_b.reshape(1, d2),
      gat_w, gat_b.reshape(1, DIM_IN), gat_out_w, gat_out_b.reshape(1, 2),
      edge_fc_w, edge_fc_b.reshape(1, 1), inv_w, inv_b.reshape(1, 2),
      mix_w, mix_b.reshape(1, 2))


def _sc_body(tab_ref, perm_ref, src_ref, dst_ref, nzE_ref, init0_ref, initz_ref,
             acc_out, att_out,
             t0, t1, t2, t3, t4, t5, acc, srcb, dstb, nzb, attb, permb,
             tmpacc, sh_up0, sh_up1, sh_acc):
    wid = lax.axis_index("s")
    nbase = pl.multiple_of(wid * NPT, 8)

    pltpu.sync_copy(tab_ref.at[pl.ds(2 * NP, NP)], t0)
    pltpu.sync_copy(tab_ref.at[pl.ds(3 * NP, NP)], t1)
    pltpu.sync_copy(tab_ref.at[pl.ds(4 * NP, NP)], t2)
    pltpu.sync_copy(tab_ref.at[pl.ds(5 * NP, NP)], t3)
    pltpu.sync_copy(tab_ref.at[pl.ds(6 * NP, NP)], t4)
    pltpu.sync_copy(tab_ref.at[pl.ds(7 * NP, NP)], t5)
    @pl.when(wid == 0)
    def _():
        pltpu.sync_copy(init0_ref, acc)

    @pl.when(wid != 0)
    def _():
        pltpu.sync_copy(initz_ref, acc)

    pltpu.sync_copy(perm_ref.at[pl.ds(nbase, NPT)], permb)

    @plsc.parallel_loop(0, NPT // L, 1, unroll=2)
    def _(j):
        p = permb[pl.ds(j * L, L)]
        g0 = plsc.load_gather(t2, [p])
        g1 = plsc.load_gather(t3, [p])
        o = nbase + j * L
        t0[pl.ds(o, L)] = t0[pl.ds(o, L)] + g0
        t1[pl.ds(o, L)] = t1[pl.ds(o, L)] + g1
    pltpu.sync_copy(t0.at[pl.ds(nbase, NPT)], sh_up0.at[pl.ds(nbase, NPT)])
    pltpu.sync_copy(t1.at[pl.ds(nbase, NPT)], sh_up1.at[pl.ds(nbase, NPT)])
    pltpu.sync_copy(tab_ref.at[pl.ds(0, NP)], t2)
    pltpu.sync_copy(tab_ref.at[pl.ds(NP, NP)], t3)
    plsc.subcore_barrier()
    pltpu.sync_copy(sh_up0, t0)
    pltpu.sync_copy(sh_up1, t1)

    for ch in range(EPW // C):
        eb = pl.multiple_of(wid * EPW + ch * C, 8)
        pltpu.sync_copy(src_ref.at[pl.ds(eb, C)], srcb)
        pltpu.sync_copy(dst_ref.at[pl.ds(eb, C)], dstb)
        pltpu.sync_copy(nzE_ref.at[pl.ds(eb, C)], nzb)

        @plsc.parallel_loop(0, C // L, 1, unroll=4)
        def _(j):
            s = srcb[pl.ds(j * L, L)]
            d = dstb[pl.ds(j * L, L)]
            esv = plsc.load_gather(t4, [s])
            edv = plsc.load_gather(t5, [d])
            lg = esv + edv + nzb[pl.ds(j * L, L)]
            att = 1.0 / (1.0 + jnp.exp(-lg))
            attb[pl.ds(j * L, L)] = att
            uc0 = plsc.load_gather(t2, [s])
            uc1 = plsc.load_gather(t3, [s])
            up0 = plsc.load_gather(t0, [s])
            up1 = plsc.load_gather(t1, [s])
            r = lax.shift_right_logical(d, 5)
            lb = lax.shift_left(d & 31, 2)
            plsc.addupdate_scatter(acc, [r, lb], att * uc0)
            plsc.addupdate_scatter(acc, [r, lb + 1], att * uc1)
            plsc.addupdate_scatter(acc, [r, lb + 2], att * up0)
            plsc.addupdate_scatter(acc, [r, lb + 3], att * up1)
        pltpu.sync_copy(attb, att_out.at[pl.ds(eb, C)])

    iota16 = lax.iota(jnp.int32, L)

    @pl.when(wid == 0)
    def _():
        pltpu.sync_copy(acc, sh_acc)

    plsc.subcore_barrier()

    @pl.when(wid != 0)
    def _():
        for g in range(NR // L):
            pltpu.sync_copy(acc.at[pl.ds(L * g, L)],
                            sh_acc.at[iota16 + L * g], add=True)

    plsc.subcore_barrier()

    @pl.when(wid < 8)
    def _():
        ob = pl.multiple_of(wid * 40, 8)
        pltpu.sync_copy(sh_acc.at[pl.ds(ob, 40)], tmpacc)
        pltpu.sync_copy(tmpacc, acc_out.at[pl.ds(ob, 40)])


def _sc_edges(table_t, perm_pad, src, dst, nz, init0, initz):
    mesh = plsc.VectorSubcoreMesh(core_axis_name="c", subcore_axis_name="s",
                                  num_cores=1)
    f32 = jnp.float32
    i32 = jnp.int32
    kfn = pl.kernel(
        _sc_body,
        compiler_params=pltpu.CompilerParams(needs_layout_passes=False),
        out_type=[
            jax.ShapeDtypeStruct((NR, 128), f32),
            jax.ShapeDtypeStruct((E,), f32),
        ],
        mesh=mesh,
        scratch_types=[
            pltpu.VMEM((NP,), f32),
            pltpu.VMEM((NP,), f32),
            pltpu.VMEM((NP,), f32),
            pltpu.VMEM((NP,), f32),
            pltpu.VMEM((NP,), f32),
            pltpu.VMEM((NP,), f32),
            pltpu.VMEM((NR, 128), f32),
            pltpu.VMEM((C,), i32),
            pltpu.VMEM((C,), i32),
            pltpu.VMEM((C,), f32),
            pltpu.VMEM((C,), f32),
            pltpu.VMEM((NPT,), i32),
            pltpu.VMEM((40, 128), f32),
            pltpu.VMEM_SHARED((NP,), f32),
            pltpu.VMEM_SHARED((NP,), f32),
            pltpu.VMEM_SHARED((NR, 128), f32),
        ],
    )
    return kfn(table_t, perm_pad, src, dst, nz, init0, initz)


def kernel(x, edge_index, tg_mask, batch, ptr,
           feature_fc_w, feature_fc_b, edge_fc_w, edge_fc_b,
           gat_w, gat_b, gat_out_w, gat_out_b,
           inv_w, inv_b, mix_w, mix_b):
    n0_pad, n1, perm_pad = _get_consts()
    f32 = jnp.float32

    feature_mask, tab_t, init0 = _tc_dense(
        x, jnp.asarray(n0_pad), feature_fc_w, feature_fc_b, gat_w, gat_b,
        gat_out_w, gat_out_b, edge_fc_w, edge_fc_b, inv_w, inv_b, mix_w, mix_b)
    table_t = tab_t.reshape(-1)
    initz = jnp.zeros((NR, 128), f32)
    acc2d, att = _sc_edges(table_t, jnp.asarray(perm_pad),
                           edge_index[0], edge_index[1], jnp.asarray(n1),
                           init0, initz)
    accr = acc2d.reshape(NP, 4)[:N]
    perm_pred = accr[:, 2:4]
    xc_pred = accr[:, 0:2]
    edge_att = att.reshape(E, 1)
    return (perm_pred, xc_pred, feature_mask, edge_att)


# speedup vs baseline: 42.3527x; 1.1101x over previous
import functools

import jax
import jax.numpy as jnp
import numpy as np
from jax import lax
from jax.experimental import pallas as pl
from jax.experimental.pallas import tpu as pltpu
from jax.experimental.pallas import tpu_sc as plsc

N = 10000
E = 320000
DIM_IN = 128

NS = 16
NP = 10240
NPT = NP // NS
NP4 = NP * 4
EPW = E // (2 * NS)
C = 2000
L = 16
NR = NP4 // 128
GR = 80
RPT = NR // NS


def _raw_consts():
    nkey = jax.random.key(42)

    def logit_noise(key, shape):
        u = jax.random.uniform(key, shape, minval=1e-10, maxval=1.0 - 1e-10,
                               dtype=jnp.float32)
        return jnp.log(u) - jnp.log(1.0 - u)

    n0 = logit_noise(jax.random.fold_in(nkey, 0), (N, DIM_IN - 2))
    n1 = logit_noise(jax.random.fold_in(nkey, 1), (E, 1))
    perm = jax.random.permutation(jax.random.fold_in(nkey, 2), N)
    return n0, n1, perm


def _consts_np():
    cpu = jax.devices("cpu")[0]
    with jax.default_device(cpu):
        n0, n1, perm = _raw_consts()
        n0, n1, perm = np.asarray(n0), np.asarray(n1), np.asarray(perm)
    n0_pad = np.zeros((NP, DIM_IN - 2), dtype=np.float32)
    n0_pad[:N] = n0
    perm_pad = np.zeros((NP,), dtype=np.int32)
    perm_pad[:N] = perm.astype(np.int32)
    return n0_pad, n1.reshape(E).astype(np.float32), perm_pad


try:
    _CONSTS = _consts_np()
except Exception:
    _CONSTS = None


def _get_consts():
    if _CONSTS is not None:
        return _CONSTS
    n0, n1, perm = _raw_consts()
    n0_pad = jnp.zeros((NP, DIM_IN - 2), jnp.float32).at[:N].set(n0)
    perm_pad = jnp.zeros((NP,), jnp.int32).at[:N].set(perm.astype(jnp.int32))
    return n0_pad, n1.reshape(E), perm_pad

_BN = 2048


def _tc_body(x_ref, nz_ref, wf_ref, bf_ref, gw_ref, gb_ref, gow_ref,
             gob_ref, efw_ref, eb_ref, ivw_ref, ivb_ref, mxw_ref, mxb_ref,
             mask_ref, tab_ref, init_ref):
    f32 = jnp.float32
    xb = x_ref[...]
    ml = jnp.dot(xb, wf_ref[...], preferred_element_type=f32)
    mask = jax.nn.sigmoid(ml + bf_ref[...] + nz_ref[...])
    mask_ref[...] = mask
    x126 = xb[:, : DIM_IN - 2]
    xl = xb[:, DIM_IN - 2:]
    y = jnp.concatenate([x126 * mask, xl], axis=1)

    ivw = ivw_ref[...]
    mxw = mxw_ref[...]
    a = jnp.dot(gw_ref[...], gow_ref[...], preferred_element_type=f32)
    wc = jnp.dot(a, ivw, preferred_element_type=f32)
    wp = jnp.dot(a, mxw, preferred_element_type=f32)
    row = lax.broadcasted_iota(jnp.int32, wp.shape, 0)
    wp_z = jnp.where(row < DIM_IN - 2, wp, 0.0)
    gba = jnp.dot(gb_ref[...], gow_ref[...], preferred_element_type=f32)
    bc = jnp.dot(gba, ivw, preferred_element_type=f32)
    bp = jnp.dot(gba, mxw, preferred_element_type=f32)
    efw = efw_ref[...]
    ews = efw[: DIM_IN]
    ewd = efw[DIM_IN:]

    uc = jnp.dot(y, wc, preferred_element_type=f32) + bc
    upb = jnp.dot(y, wp, preferred_element_type=f32) + bp
    v = jnp.dot(xb - y, wp_z, preferred_element_type=f32)
    es = jnp.dot(xb, ews, preferred_element_type=f32) + eb_ref[...]
    ed = jnp.dot(xb, ewd, preferred_element_type=f32)
    tab = jnp.concatenate([uc, upb, v, es, ed], axis=1)
    tab_ref[...] = tab.T

    gob = gob_ref[...]
    ccc = jnp.dot(gob, ivw, preferred_element_type=f32) + ivb_ref[...]
    ccp = jnp.dot(gob, mxw, preferred_element_type=f32) + mxb_ref[...]
    cv4 = jnp.concatenate([ccc, ccp], axis=1)
    init_ref[...] = jnp.tile(cv4, (NR, 32))


def _tc_dense(x, nz_pad, feature_fc_w, feature_fc_b, gat_w, gat_b,
              gat_out_w, gat_out_b, edge_fc_w, edge_fc_b,
              inv_w, inv_b, mix_w, mix_b, interpret=False):
    grid = (NP // _BN,)
    d2 = DIM_IN - 2
    full = lambda shape: pl.BlockSpec(shape, lambda i: tuple(0 for _ in shape))
    return pl.pallas_call(
        _tc_body,
        grid=grid,
        in_specs=[
            pl.BlockSpec((_BN, DIM_IN), lambda i: (i, 0)),
            pl.BlockSpec((_BN, d2), lambda i: (i, 0)),
            full((DIM_IN, d2)),
            full((1, d2)),
            full((DIM_IN, DIM_IN)),
            full((1, DIM_IN)),
            full((DIM_IN, 2)),
            full((1, 2)),
            full((2 * DIM_IN, 1)),
            full((1, 1)),
            full((2, 2)),
            full((1, 2)),
            full((2, 2)),
            full((1, 2)),
        ],
        out_specs=[
            pl.BlockSpec((_BN, d2), lambda i: (i, 0)),
            pl.BlockSpec((8, _BN), lambda i: (0, i)),
            pl.BlockSpec((NR, 128), lambda i: (0, 0)),
        ],
        out_shape=[
            jax.ShapeDtypeStruct((N, d2), jnp.float32),
            jax.ShapeDtypeStruct((8, NP), jnp.float32),
            jax.ShapeDtypeStruct((NR, 128), jnp.float32),
        ],
        interpret=interpret,
    )(x, nz_pad, feature_fc_w, feature_fc_b.reshape(1, d2),
      gat_w, gat_b.reshape(1, DIM_IN), gat_out_w, gat_out_b.reshape(1, 2),
      edge_fc_w, edge_fc_b.reshape(1, 1), inv_w, inv_b.reshape(1, 2),
      mix_w, mix_b.reshape(1, 2))


def _sc_body(tab_ref, perm_ref, src_ref, dst_ref, nzE_ref, init0_ref, initz_ref,
             acc_out, att_out,
             t0, t1, t2, t3, t4, t5, acc, srcb, dstb, nzb, attb, permb,
             tmpacc, sh_up0, sh_up1, sh_acc):
    cid = lax.axis_index("c")
    wid = lax.axis_index("s")
    ew = cid * NS + wid
    nbase = pl.multiple_of(wid * NPT, 8)

    pltpu.sync_copy(tab_ref.at[pl.ds(2 * NP, NP)], t0)
    pltpu.sync_copy(tab_ref.at[pl.ds(3 * NP, NP)], t1)
    pltpu.sync_copy(tab_ref.at[pl.ds(4 * NP, NP)], t2)
    pltpu.sync_copy(tab_ref.at[pl.ds(5 * NP, NP)], t3)
    pltpu.sync_copy(tab_ref.at[pl.ds(6 * NP, NP)], t4)
    pltpu.sync_copy(tab_ref.at[pl.ds(7 * NP, NP)], t5)
    @pl.when(ew == 0)
    def _():
        pltpu.sync_copy(init0_ref, acc)

    @pl.when(ew != 0)
    def _():
        pltpu.sync_copy(initz_ref, acc)

    pltpu.sync_copy(perm_ref.at[pl.ds(nbase, NPT)], permb)

    @plsc.parallel_loop(0, NPT // L, 1, unroll=2)
    def _(j):
        p = permb[pl.ds(j * L, L)]
        g0 = plsc.load_gather(t2, [p])
        g1 = plsc.load_gather(t3, [p])
        o = nbase + j * L
        t0[pl.ds(o, L)] = t0[pl.ds(o, L)] + g0
        t1[pl.ds(o, L)] = t1[pl.ds(o, L)] + g1
    pltpu.sync_copy(t0.at[pl.ds(nbase, NPT)], sh_up0.at[pl.ds(nbase, NPT)])
    pltpu.sync_copy(t1.at[pl.ds(nbase, NPT)], sh_up1.at[pl.ds(nbase, NPT)])
    pltpu.sync_copy(tab_ref.at[pl.ds(0, NP)], t2)
    pltpu.sync_copy(tab_ref.at[pl.ds(NP, NP)], t3)
    plsc.subcore_barrier()
    pltpu.sync_copy(sh_up0, t0)
    pltpu.sync_copy(sh_up1, t1)

    for ch in range(EPW // C):
        eb = pl.multiple_of(ew * EPW + ch * C, 8)
        pltpu.sync_copy(src_ref.at[pl.ds(eb, C)], srcb)
        pltpu.sync_copy(dst_ref.at[pl.ds(eb, C)], dstb)
        pltpu.sync_copy(nzE_ref.at[pl.ds(eb, C)], nzb)

        @plsc.parallel_loop(0, C // L, 1, unroll=4)
        def _(j):
            s = srcb[pl.ds(j * L, L)]
            d = dstb[pl.ds(j * L, L)]
            esv = plsc.load_gather(t4, [s])
            edv = plsc.load_gather(t5, [d])
            lg = esv + edv + nzb[pl.ds(j * L, L)]
            att = 1.0 / (1.0 + jnp.exp(-lg))
            attb[pl.ds(j * L, L)] = att
            uc0 = plsc.load_gather(t2, [s])
            uc1 = plsc.load_gather(t3, [s])
            up0 = plsc.load_gather(t0, [s])
            up1 = plsc.load_gather(t1, [s])
            r = lax.shift_right_logical(d, 5)
            lb = lax.shift_left(d & 31, 2)
            plsc.addupdate_scatter(acc, [r, lb], att * uc0)
            plsc.addupdate_scatter(acc, [r, lb + 1], att * uc1)
            plsc.addupdate_scatter(acc, [r, lb + 2], att * up0)
            plsc.addupdate_scatter(acc, [r, lb + 3], att * up1)
        pltpu.sync_copy(attb, att_out.at[pl.ds(eb, C)])

    iota16 = lax.iota(jnp.int32, L)

    @pl.when(wid == 0)
    def _():
        pltpu.sync_copy(acc, sh_acc)

    plsc.subcore_barrier()

    @pl.when(wid != 0)
    def _():
        for g in range(NR // L):
            pltpu.sync_copy(acc.at[pl.ds(L * g, L)],
                            sh_acc.at[iota16 + L * g], add=True)

    plsc.subcore_barrier()

    @pl.when(wid < 8)
    def _():
        ob = pl.multiple_of(wid * 40, 8)
        pltpu.sync_copy(sh_acc.at[pl.ds(ob, 40)], tmpacc)
        pltpu.sync_copy(tmpacc, acc_out.at[cid, pl.ds(ob, 40)])


def _sc_edges(table_t, perm_pad, src, dst, nz, init0, initz):
    mesh = plsc.VectorSubcoreMesh(core_axis_name="c", subcore_axis_name="s",
                                  num_cores=2)
    f32 = jnp.float32
    i32 = jnp.int32
    kfn = pl.kernel(
        _sc_body,
        compiler_params=pltpu.CompilerParams(needs_layout_passes=False),
        out_type=[
            jax.ShapeDtypeStruct((2, NR, 128), f32),
            jax.ShapeDtypeStruct((E,), f32),
        ],
        mesh=mesh,
        scratch_types=[
            pltpu.VMEM((NP,), f32),
            pltpu.VMEM((NP,), f32),
            pltpu.VMEM((NP,), f32),
            pltpu.VMEM((NP,), f32),
            pltpu.VMEM((NP,), f32),
            pltpu.VMEM((NP,), f32),
            pltpu.VMEM((NR, 128), f32),
            pltpu.VMEM((C,), i32),
            pltpu.VMEM((C,), i32),
            pltpu.VMEM((C,), f32),
            pltpu.VMEM((C,), f32),
            pltpu.VMEM((NPT,), i32),
            pltpu.VMEM((40, 128), f32),
            pltpu.VMEM_SHARED((NP,), f32),
            pltpu.VMEM_SHARED((NP,), f32),
            pltpu.VMEM_SHARED((NR, 128), f32),
        ],
    )
    return kfn(table_t, perm_pad, src, dst, nz, init0, initz)


def _tc_merge_body(p_ref, out_ref):
    out_ref[...] = p_ref[0] + p_ref[1]


def _tc_merge(parts):
    return pl.pallas_call(
        _tc_merge_body,
        out_shape=jax.ShapeDtypeStruct((NR, 128), jnp.float32),
    )(parts)


def kernel(x, edge_index, tg_mask, batch, ptr,
           feature_fc_w, feature_fc_b, edge_fc_w, edge_fc_b,
           gat_w, gat_b, gat_out_w, gat_out_b,
           inv_w, inv_b, mix_w, mix_b):
    n0_pad, n1, perm_pad = _get_consts()
    f32 = jnp.float32

    feature_mask, tab_t, init0 = _tc_dense(
        x, jnp.asarray(n0_pad), feature_fc_w, feature_fc_b, gat_w, gat_b,
        gat_out_w, gat_out_b, edge_fc_w, edge_fc_b, inv_w, inv_b, mix_w, mix_b)
    table_t = tab_t.reshape(-1)
    initz = jnp.zeros((NR, 128), f32)
    parts, att = _sc_edges(table_t, jnp.asarray(perm_pad),
                           edge_index[0], edge_index[1], jnp.asarray(n1),
                           init0, initz)
    acc2d = _tc_merge(parts)
    accr = acc2d.reshape(NP, 4)[:N]
    perm_pred = accr[:, 2:4]
    xc_pred = accr[:, 0:2]
    edge_att = att.reshape(E, 1)
    return (perm_pred, xc_pred, feature_mask, edge_att)


# speedup vs baseline: 42.5210x; 1.0040x over previous
import functools

import jax
import jax.numpy as jnp
import numpy as np
from jax import lax
from jax.experimental import pallas as pl
from jax.experimental.pallas import tpu as pltpu
from jax.experimental.pallas import tpu_sc as plsc

N = 10000
E = 320000
DIM_IN = 128

NS = 16
NP = 10240
NPT = NP // NS
NP4 = NP * 4
EPW = E // (2 * NS)
C = 2000
L = 16
NR = NP4 // 128
GR = 80
RPT = NR // NS


def _raw_consts():
    nkey = jax.random.key(42)

    def logit_noise(key, shape):
        u = jax.random.uniform(key, shape, minval=1e-10, maxval=1.0 - 1e-10,
                               dtype=jnp.float32)
        return jnp.log(u) - jnp.log(1.0 - u)

    n0 = logit_noise(jax.random.fold_in(nkey, 0), (N, DIM_IN - 2))
    n1 = logit_noise(jax.random.fold_in(nkey, 1), (E, 1))
    perm = jax.random.permutation(jax.random.fold_in(nkey, 2), N)
    return n0, n1, perm


def _consts_np():
    cpu = jax.devices("cpu")[0]
    with jax.default_device(cpu):
        n0, n1, perm = _raw_consts()
        n0, n1, perm = np.asarray(n0), np.asarray(n1), np.asarray(perm)
    n0_pad = np.zeros((NP, DIM_IN - 2), dtype=np.float32)
    n0_pad[:N] = n0
    perm_pad = np.zeros((NP,), dtype=np.int32)
    perm_pad[:N] = perm.astype(np.int32)
    return n0_pad, n1.reshape(E).astype(np.float32), perm_pad


try:
    _CONSTS = _consts_np()
except Exception:
    _CONSTS = None


def _get_consts():
    if _CONSTS is not None:
        return _CONSTS
    n0, n1, perm = _raw_consts()
    n0_pad = jnp.zeros((NP, DIM_IN - 2), jnp.float32).at[:N].set(n0)
    perm_pad = jnp.zeros((NP,), jnp.int32).at[:N].set(perm.astype(jnp.int32))
    return n0_pad, n1.reshape(E), perm_pad

_BN = 2048


def _tc_body(x_ref, nz_ref, wf_ref, bf_ref, gw_ref, gb_ref, gow_ref,
             gob_ref, efw_ref, eb_ref, ivw_ref, ivb_ref, mxw_ref, mxb_ref,
             mask_ref, tab_ref, init_ref):
    f32 = jnp.float32
    xb = x_ref[...]
    ml = jnp.dot(xb, wf_ref[...], preferred_element_type=f32)
    mask = jax.nn.sigmoid(ml + bf_ref[...] + nz_ref[...])
    mask_ref[...] = mask
    x126 = xb[:, : DIM_IN - 2]
    xl = xb[:, DIM_IN - 2:]
    y = jnp.concatenate([x126 * mask, xl], axis=1)

    ivw = ivw_ref[...]
    mxw = mxw_ref[...]
    a = jnp.dot(gw_ref[...], gow_ref[...], preferred_element_type=f32)
    wc = jnp.dot(a, ivw, preferred_element_type=f32)
    wp = jnp.dot(a, mxw, preferred_element_type=f32)
    row = lax.broadcasted_iota(jnp.int32, wp.shape, 0)
    wp_z = jnp.where(row < DIM_IN - 2, wp, 0.0)
    gba = jnp.dot(gb_ref[...], gow_ref[...], preferred_element_type=f32)
    bc = jnp.dot(gba, ivw, preferred_element_type=f32)
    bp = jnp.dot(gba, mxw, preferred_element_type=f32)
    efw = efw_ref[...]
    ews = efw[: DIM_IN]
    ewd = efw[DIM_IN:]

    uc = jnp.dot(y, wc, preferred_element_type=f32) + bc
    upb = jnp.dot(y, wp, preferred_element_type=f32) + bp
    v = jnp.dot(xb - y, wp_z, preferred_element_type=f32)
    es = jnp.dot(xb, ews, preferred_element_type=f32) + eb_ref[...]
    ed = jnp.dot(xb, ewd, preferred_element_type=f32)
    tab = jnp.concatenate([uc, upb, v, es, ed], axis=1)
    tab_ref[...] = tab.T

    gob = gob_ref[...]
    ccc = jnp.dot(gob, ivw, preferred_element_type=f32) + ivb_ref[...]
    ccp = jnp.dot(gob, mxw, preferred_element_type=f32) + mxb_ref[...]
    cv4 = jnp.concatenate([ccc, ccp], axis=1)
    init_ref[...] = jnp.tile(cv4, (NR, 32))


def _tc_dense(x, nz_pad, feature_fc_w, feature_fc_b, gat_w, gat_b,
              gat_out_w, gat_out_b, edge_fc_w, edge_fc_b,
              inv_w, inv_b, mix_w, mix_b, interpret=False):
    grid = (NP // _BN,)
    d2 = DIM_IN - 2
    full = lambda shape: pl.BlockSpec(shape, lambda i: tuple(0 for _ in shape))
    return pl.pallas_call(
        _tc_body,
        grid=grid,
        in_specs=[
            pl.BlockSpec((_BN, DIM_IN), lambda i: (i, 0)),
            pl.BlockSpec((_BN, d2), lambda i: (i, 0)),
            full((DIM_IN, d2)),
            full((1, d2)),
            full((DIM_IN, DIM_IN)),
            full((1, DIM_IN)),
            full((DIM_IN, 2)),
            full((1, 2)),
            full((2 * DIM_IN, 1)),
            full((1, 1)),
            full((2, 2)),
            full((1, 2)),
            full((2, 2)),
            full((1, 2)),
        ],
        out_specs=[
            pl.BlockSpec((_BN, d2), lambda i: (i, 0)),
            pl.BlockSpec((8, _BN), lambda i: (0, i)),
            pl.BlockSpec((NR, 128), lambda i: (0, 0)),
        ],
        out_shape=[
            jax.ShapeDtypeStruct((N, d2), jnp.float32),
            jax.ShapeDtypeStruct((8, NP), jnp.float32),
            jax.ShapeDtypeStruct((NR, 128), jnp.float32),
        ],
        interpret=interpret,
    )(x, nz_pad, feature_fc_w, feature_fc_b.reshape(1, d2),
      gat_w, gat_b.reshape(1, DIM_IN), gat_out_w, gat_out_b.reshape(1, 2),
      edge_fc_w, edge_fc_b.reshape(1, 1), inv_w, inv_b.reshape(1, 2),
      mix_w, mix_b.reshape(1, 2))


def _sc_body(tab_ref, perm_ref, src_ref, dst_ref, nzE_ref, init0_ref,
             acc_out, att_out,
             t0, t1, t2, t3, t4, t5, acc, srcb, dstb, nzb, attb, permb,
             tmpacc, sh_up0, sh_up1, sh_acc):
    cid = lax.axis_index("c")
    wid = lax.axis_index("s")
    ew = cid * NS + wid
    nbase = pl.multiple_of(wid * NPT, 8)

    pltpu.sync_copy(tab_ref.at[pl.ds(2 * NP, NP)], t0)
    pltpu.sync_copy(tab_ref.at[pl.ds(3 * NP, NP)], t1)
    pltpu.sync_copy(tab_ref.at[pl.ds(4 * NP, NP)], t2)
    pltpu.sync_copy(tab_ref.at[pl.ds(5 * NP, NP)], t3)
    pltpu.sync_copy(tab_ref.at[pl.ds(6 * NP, NP)], t4)
    pltpu.sync_copy(tab_ref.at[pl.ds(7 * NP, NP)], t5)
    @pl.when(ew == 0)
    def _():
        pltpu.sync_copy(init0_ref, acc)

    @pl.when(ew != 0)
    def _():
        zv = jnp.zeros((L,), jnp.float32)

        @plsc.parallel_loop(0, NR, 1, unroll=2)
        def _(i):
            for jj in range(128 // L):
                acc[i, pl.ds(jj * L, L)] = zv

    pltpu.sync_copy(perm_ref.at[pl.ds(nbase, NPT)], permb)

    @plsc.parallel_loop(0, NPT // L, 1, unroll=2)
    def _(j):
        p = permb[pl.ds(j * L, L)]
        g0 = plsc.load_gather(t2, [p])
        g1 = plsc.load_gather(t3, [p])
        o = nbase + j * L
        t0[pl.ds(o, L)] = t0[pl.ds(o, L)] + g0
        t1[pl.ds(o, L)] = t1[pl.ds(o, L)] + g1
    pltpu.sync_copy(t0.at[pl.ds(nbase, NPT)], sh_up0.at[pl.ds(nbase, NPT)])
    pltpu.sync_copy(t1.at[pl.ds(nbase, NPT)], sh_up1.at[pl.ds(nbase, NPT)])
    pltpu.sync_copy(tab_ref.at[pl.ds(0, NP)], t2)
    pltpu.sync_copy(tab_ref.at[pl.ds(NP, NP)], t3)
    plsc.subcore_barrier()
    pltpu.sync_copy(sh_up0, t0)
    pltpu.sync_copy(sh_up1, t1)

    for ch in range(EPW // C):
        eb = pl.multiple_of(ew * EPW + ch * C, 8)
        pltpu.sync_copy(src_ref.at[pl.ds(eb, C)], srcb)
        pltpu.sync_copy(dst_ref.at[pl.ds(eb, C)], dstb)
        pltpu.sync_copy(nzE_ref.at[pl.ds(eb, C)], nzb)

        @plsc.parallel_loop(0, C // L, 1, unroll=8)
        def _(j):
            s = srcb[pl.ds(j * L, L)]
            d = dstb[pl.ds(j * L, L)]
            esv = plsc.load_gather(t4, [s])
            edv = plsc.load_gather(t5, [d])
            lg = esv + edv + nzb[pl.ds(j * L, L)]
            att = 1.0 / (1.0 + jnp.exp(-lg))
            attb[pl.ds(j * L, L)] = att
            uc0 = plsc.load_gather(t2, [s])
            uc1 = plsc.load_gather(t3, [s])
            up0 = plsc.load_gather(t0, [s])
            up1 = plsc.load_gather(t1, [s])
            r = lax.shift_right_logical(d, 5)
            lb = lax.shift_left(d & 31, 2)
            plsc.addupdate_scatter(acc, [r, lb], att * uc0)
            plsc.addupdate_scatter(acc, [r, lb + 1], att * uc1)
            plsc.addupdate_scatter(acc, [r, lb + 2], att * up0)
            plsc.addupdate_scatter(acc, [r, lb + 3], att * up1)
        pltpu.sync_copy(attb, att_out.at[pl.ds(eb, C)])

    iota16 = lax.iota(jnp.int32, L)

    @pl.when(wid == 0)
    def _():
        pltpu.sync_copy(acc, sh_acc)

    plsc.subcore_barrier()

    @pl.when(wid != 0)
    def _():
        for g in range(NR // L):
            pltpu.sync_copy(acc.at[pl.ds(L * g, L)],
                            sh_acc.at[iota16 + L * g], add=True)

    plsc.subcore_barrier()

    @pl.when(wid < 8)
    def _():
        ob = pl.multiple_of(wid * 40, 8)
        pltpu.sync_copy(sh_acc.at[pl.ds(ob, 40)], tmpacc)
        pltpu.sync_copy(tmpacc, acc_out.at[cid, pl.ds(ob, 40)])


def _sc_edges(table_t, perm_pad, src, dst, nz, init0):
    mesh = plsc.VectorSubcoreMesh(core_axis_name="c", subcore_axis_name="s",
                                  num_cores=2)
    f32 = jnp.float32
    i32 = jnp.int32
    kfn = pl.kernel(
        _sc_body,
        compiler_params=pltpu.CompilerParams(needs_layout_passes=False),
        out_type=[
            jax.ShapeDtypeStruct((2, NR, 128), f32),
            jax.ShapeDtypeStruct((E,), f32),
        ],
        mesh=mesh,
        scratch_types=[
            pltpu.VMEM((NP,), f32),
            pltpu.VMEM((NP,), f32),
            pltpu.VMEM((NP,), f32),
            pltpu.VMEM((NP,), f32),
            pltpu.VMEM((NP,), f32),
            pltpu.VMEM((NP,), f32),
            pltpu.VMEM((NR, 128), f32),
            pltpu.VMEM((C,), i32),
            pltpu.VMEM((C,), i32),
            pltpu.VMEM((C,), f32),
            pltpu.VMEM((C,), f32),
            pltpu.VMEM((NPT,), i32),
            pltpu.VMEM((40, 128), f32),
            pltpu.VMEM_SHARED((NP,), f32),
            pltpu.VMEM_SHARED((NP,), f32),
            pltpu.VMEM_SHARED((NR, 128), f32),
        ],
    )
    return kfn(table_t, perm_pad, src, dst, nz, init0)


def _tc_merge_body(p_ref, out_ref):
    out_ref[...] = p_ref[0] + p_ref[1]


def _tc_merge(parts):
    return pl.pallas_call(
        _tc_merge_body,
        out_shape=jax.ShapeDtypeStruct((NR, 128), jnp.float32),
    )(parts)


def kernel(x, edge_index, tg_mask, batch, ptr,
           feature_fc_w, feature_fc_b, edge_fc_w, edge_fc_b,
           gat_w, gat_b, gat_out_w, gat_out_b,
           inv_w, inv_b, mix_w, mix_b):
    n0_pad, n1, perm_pad = _get_consts()
    f32 = jnp.float32

    feature_mask, tab_t, init0 = _tc_dense(
        x, jnp.asarray(n0_pad), feature_fc_w, feature_fc_b, gat_w, gat_b,
        gat_out_w, gat_out_b, edge_fc_w, edge_fc_b, inv_w, inv_b, mix_w, mix_b)
    table_t = tab_t.reshape(-1)
    parts, att = _sc_edges(table_t, jnp.asarray(perm_pad),
                           edge_index[0], edge_index[1], jnp.asarray(n1),
                           init0)
    acc2d = _tc_merge(parts)
    accr = acc2d.reshape(NP, 4)[:N]
    perm_pred = accr[:, 2:4]
    xc_pred = accr[:, 0:2]
    edge_att = att.reshape(E, 1)
    return (perm_pred, xc_pred, feature_mask, edge_att)


# speedup vs baseline: 42.9545x; 1.0102x over previous
import functools

import jax
import jax.numpy as jnp
import numpy as np
from jax import lax
from jax.experimental import pallas as pl
from jax.experimental.pallas import tpu as pltpu
from jax.experimental.pallas import tpu_sc as plsc

N = 10000
E = 320000
DIM_IN = 128

NS = 16
NP = 10240
NPT = NP // NS
NP4 = NP * 4
EPW = E // (2 * NS)
C = 2000
L = 16
NR = NP4 // 128
GR = 80
RPT = NR // NS


def _raw_consts():
    nkey = jax.random.key(42)

    def logit_noise(key, shape):
        u = jax.random.uniform(key, shape, minval=1e-10, maxval=1.0 - 1e-10,
                               dtype=jnp.float32)
        return jnp.log(u) - jnp.log(1.0 - u)

    n0 = logit_noise(jax.random.fold_in(nkey, 0), (N, DIM_IN - 2))
    n1 = logit_noise(jax.random.fold_in(nkey, 1), (E, 1))
    perm = jax.random.permutation(jax.random.fold_in(nkey, 2), N)
    return n0, n1, perm


def _consts_np():
    cpu = jax.devices("cpu")[0]
    with jax.default_device(cpu):
        n0, n1, perm = _raw_consts()
        n0, n1, perm = np.asarray(n0), np.asarray(n1), np.asarray(perm)
    n0_pad = np.zeros((NP, DIM_IN - 2), dtype=np.float32)
    n0_pad[:N] = n0
    perm_pad = np.zeros((NP,), dtype=np.int32)
    perm_pad[:N] = perm.astype(np.int32)
    return n0_pad, n1.reshape(E).astype(np.float32), perm_pad


try:
    _CONSTS = _consts_np()
except Exception:
    _CONSTS = None


def _get_consts():
    if _CONSTS is not None:
        return _CONSTS
    n0, n1, perm = _raw_consts()
    n0_pad = jnp.zeros((NP, DIM_IN - 2), jnp.float32).at[:N].set(n0)
    perm_pad = jnp.zeros((NP,), jnp.int32).at[:N].set(perm.astype(jnp.int32))
    return n0_pad, n1.reshape(E), perm_pad

_BN = 2048


def _tc_body(x_ref, nz_ref, wf_ref, bf_ref, gw_ref, gb_ref, gow_ref,
             gob_ref, efw_ref, eb_ref, ivw_ref, ivb_ref, mxw_ref, mxb_ref,
             mask_ref, tab_ref, init_ref):
    f32 = jnp.float32
    xb = x_ref[...]
    ml = jnp.dot(xb, wf_ref[...], preferred_element_type=f32)
    mask = jax.nn.sigmoid(ml + bf_ref[...] + nz_ref[...])
    mask_ref[...] = mask
    x126 = xb[:, : DIM_IN - 2]
    xl = xb[:, DIM_IN - 2:]
    y = jnp.concatenate([x126 * mask, xl], axis=1)

    ivw = ivw_ref[...]
    mxw = mxw_ref[...]
    a = jnp.dot(gw_ref[...], gow_ref[...], preferred_element_type=f32)
    wc = jnp.dot(a, ivw, preferred_element_type=f32)
    wp = jnp.dot(a, mxw, preferred_element_type=f32)
    row = lax.broadcasted_iota(jnp.int32, wp.shape, 0)
    wp_z = jnp.where(row < DIM_IN - 2, wp, 0.0)
    gba = jnp.dot(gb_ref[...], gow_ref[...], preferred_element_type=f32)
    bc = jnp.dot(gba, ivw, preferred_element_type=f32)
    bp = jnp.dot(gba, mxw, preferred_element_type=f32)
    efw = efw_ref[...]
    ews = efw[: DIM_IN]
    ewd = efw[DIM_IN:]

    uc = jnp.dot(y, wc, preferred_element_type=f32) + bc
    upb = jnp.dot(y, wp, preferred_element_type=f32) + bp
    v = jnp.dot(xb - y, wp_z, preferred_element_type=f32)
    es = jnp.dot(xb, ews, preferred_element_type=f32) + eb_ref[...]
    ed = jnp.dot(xb, ewd, preferred_element_type=f32)
    tab = jnp.concatenate([uc, upb, v, es, ed], axis=1)
    tab_ref[...] = tab.T

    gob = gob_ref[...]
    ccc = jnp.dot(gob, ivw, preferred_element_type=f32) + ivb_ref[...]
    ccp = jnp.dot(gob, mxw, preferred_element_type=f32) + mxb_ref[...]
    cv4 = jnp.concatenate([ccc, ccp], axis=1)
    init_ref[...] = jnp.tile(cv4, (NR, 32))


def _tc_dense(x, nz_pad, feature_fc_w, feature_fc_b, gat_w, gat_b,
              gat_out_w, gat_out_b, edge_fc_w, edge_fc_b,
              inv_w, inv_b, mix_w, mix_b, interpret=False):
    grid = (NP // _BN,)
    d2 = DIM_IN - 2
    full = lambda shape: pl.BlockSpec(shape, lambda i: tuple(0 for _ in shape))
    return pl.pallas_call(
        _tc_body,
        grid=grid,
        in_specs=[
            pl.BlockSpec((_BN, DIM_IN), lambda i: (i, 0)),
            pl.BlockSpec((_BN, d2), lambda i: (i, 0)),
            full((DIM_IN, d2)),
            full((1, d2)),
            full((DIM_IN, DIM_IN)),
            full((1, DIM_IN)),
            full((DIM_IN, 2)),
            full((1, 2)),
            full((2 * DIM_IN, 1)),
            full((1, 1)),
            full((2, 2)),
            full((1, 2)),
            full((2, 2)),
            full((1, 2)),
        ],
        out_specs=[
            pl.BlockSpec((_BN, d2), lambda i: (i, 0)),
            pl.BlockSpec((8, _BN), lambda i: (0, i)),
            pl.BlockSpec((NR, 128), lambda i: (0, 0)),
        ],
        out_shape=[
            jax.ShapeDtypeStruct((N, d2), jnp.float32),
            jax.ShapeDtypeStruct((8, NP), jnp.float32),
            jax.ShapeDtypeStruct((NR, 128), jnp.float32),
        ],
        interpret=interpret,
    )(x, nz_pad, feature_fc_w, feature_fc_b.reshape(1, d2),
      gat_w, gat_b.reshape(1, DIM_IN), gat_out_w, gat_out_b.reshape(1, 2),
      edge_fc_w, edge_fc_b.reshape(1, 1), inv_w, inv_b.reshape(1, 2),
      mix_w, mix_b.reshape(1, 2))


def _sc_body(tab_ref, perm_ref, src_ref, dst_ref, nzE_ref, init0_ref,
             acc_out, att_out,
             t0, t1, t2, t3, t4, t5, acc, srcb, dstb, nzb, attb, permb,
             tmpacc, idxb, sem, sh_up0, sh_up1, sh_acc):
    cid = lax.axis_index("c")
    wid = lax.axis_index("s")
    ew = cid * NS + wid
    nbase = pl.multiple_of(wid * NPT, 8)

    cps = [
        pltpu.async_copy(tab_ref.at[pl.ds(2 * NP, NP)], t0, sem),
        pltpu.async_copy(tab_ref.at[pl.ds(3 * NP, NP)], t1, sem),
        pltpu.async_copy(tab_ref.at[pl.ds(4 * NP, NP)], t2, sem),
        pltpu.async_copy(tab_ref.at[pl.ds(5 * NP, NP)], t3, sem),
        pltpu.async_copy(tab_ref.at[pl.ds(6 * NP, NP)], t4, sem),
        pltpu.async_copy(tab_ref.at[pl.ds(7 * NP, NP)], t5, sem),
        pltpu.async_copy(perm_ref.at[pl.ds(nbase, NPT)], permb, sem),
    ]

    @pl.when(ew == 0)
    def _():
        pltpu.sync_copy(init0_ref, acc)

    @pl.when(ew != 0)
    def _():
        zv = jnp.zeros((L,), jnp.float32)

        @plsc.parallel_loop(0, NR, 1, unroll=2)
        def _(i):
            for jj in range(128 // L):
                acc[i, pl.ds(jj * L, L)] = zv

    iota16 = lax.iota(jnp.int32, L)
    for g in range(NR // L):
        idxb[pl.ds(g * L, L)] = iota16 + L * g

    for cp in cps:
        cp.wait()

    @plsc.parallel_loop(0, NPT // L, 1, unroll=2)
    def _(j):
        p = permb[pl.ds(j * L, L)]
        g0 = plsc.load_gather(t2, [p])
        g1 = plsc.load_gather(t3, [p])
        o = nbase + j * L
        t0[pl.ds(o, L)] = t0[pl.ds(o, L)] + g0
        t1[pl.ds(o, L)] = t1[pl.ds(o, L)] + g1
    pltpu.sync_copy(t0.at[pl.ds(nbase, NPT)], sh_up0.at[pl.ds(nbase, NPT)])
    pltpu.sync_copy(t1.at[pl.ds(nbase, NPT)], sh_up1.at[pl.ds(nbase, NPT)])
    pltpu.sync_copy(tab_ref.at[pl.ds(0, NP)], t2)
    pltpu.sync_copy(tab_ref.at[pl.ds(NP, NP)], t3)
    plsc.subcore_barrier()
    pltpu.sync_copy(sh_up0, t0)
    pltpu.sync_copy(sh_up1, t1)

    for ch in range(EPW // C):
        eb = pl.multiple_of(ew * EPW + ch * C, 8)
        pltpu.sync_copy(src_ref.at[pl.ds(eb, C)], srcb)
        pltpu.sync_copy(dst_ref.at[pl.ds(eb, C)], dstb)
        pltpu.sync_copy(nzE_ref.at[pl.ds(eb, C)], nzb)

        @plsc.parallel_loop(0, C // L, 1, unroll=8)
        def _(j):
            s = srcb[pl.ds(j * L, L)]
            d = dstb[pl.ds(j * L, L)]
            esv = plsc.load_gather(t4, [s])
            edv = plsc.load_gather(t5, [d])
            lg = esv + edv + nzb[pl.ds(j * L, L)]
            att = 1.0 / (1.0 + jnp.exp(-lg))
            attb[pl.ds(j * L, L)] = att
            uc0 = plsc.load_gather(t2, [s])
            uc1 = plsc.load_gather(t3, [s])
            up0 = plsc.load_gather(t0, [s])
            up1 = plsc.load_gather(t1, [s])
            r = lax.shift_right_logical(d, 5)
            lb = lax.shift_left(d & 31, 2)
            plsc.addupdate_scatter(acc, [r, lb], att * uc0)
            plsc.addupdate_scatter(acc, [r, lb + 1], att * uc1)
            plsc.addupdate_scatter(acc, [r, lb + 2], att * up0)
            plsc.addupdate_scatter(acc, [r, lb + 3], att * up1)
        pltpu.sync_copy(attb, att_out.at[pl.ds(eb, C)])

    @pl.when(wid == 0)
    def _():
        pltpu.sync_copy(acc, sh_acc)

    plsc.subcore_barrier()

    @pl.when(wid != 0)
    def _():
        pltpu.sync_copy(acc, sh_acc.at[idxb], add=True)

    plsc.subcore_barrier()

    @pl.when(wid < 8)
    def _():
        ob = pl.multiple_of(wid * 40, 8)
        pltpu.sync_copy(sh_acc.at[pl.ds(ob, 40)], tmpacc)
        pltpu.sync_copy(tmpacc, acc_out.at[cid, pl.ds(ob, 40)])


def _sc_edges(table_t, perm_pad, src, dst, nz, init0):
    mesh = plsc.VectorSubcoreMesh(core_axis_name="c", subcore_axis_name="s",
                                  num_cores=2)
    f32 = jnp.float32
    i32 = jnp.int32
    kfn = pl.kernel(
        _sc_body,
        compiler_params=pltpu.CompilerParams(needs_layout_passes=False),
        out_type=[
            jax.ShapeDtypeStruct((2, NR, 128), f32),
            jax.ShapeDtypeStruct((E,), f32),
        ],
        mesh=mesh,
        scratch_types=[
            pltpu.VMEM((NP,), f32),
            pltpu.VMEM((NP,), f32),
            pltpu.VMEM((NP,), f32),
            pltpu.VMEM((NP,), f32),
            pltpu.VMEM((NP,), f32),
            pltpu.VMEM((NP,), f32),
            pltpu.VMEM((NR, 128), f32),
            pltpu.VMEM((C,), i32),
            pltpu.VMEM((C,), i32),
            pltpu.VMEM((C,), f32),
            pltpu.VMEM((C,), f32),
            pltpu.VMEM((NPT,), i32),
            pltpu.VMEM((40, 128), f32),
            pltpu.VMEM((NR,), i32),
            pltpu.SemaphoreType.DMA,
            pltpu.VMEM_SHARED((NP,), f32),
            pltpu.VMEM_SHARED((NP,), f32),
            pltpu.VMEM_SHARED((NR, 128), f32),
        ],
    )
    return kfn(table_t, perm_pad, src, dst, nz, init0)


def _tc_merge_body(p_ref, out_ref):
    out_ref[...] = p_ref[0] + p_ref[1]


def _tc_merge(parts):
    return pl.pallas_call(
        _tc_merge_body,
        out_shape=jax.ShapeDtypeStruct((NR, 128), jnp.float32),
    )(parts)


def kernel(x, edge_index, tg_mask, batch, ptr,
           feature_fc_w, feature_fc_b, edge_fc_w, edge_fc_b,
           gat_w, gat_b, gat_out_w, gat_out_b,
           inv_w, inv_b, mix_w, mix_b):
    n0_pad, n1, perm_pad = _get_consts()
    f32 = jnp.float32

    feature_mask, tab_t, init0 = _tc_dense(
        x, jnp.asarray(n0_pad), feature_fc_w, feature_fc_b, gat_w, gat_b,
        gat_out_w, gat_out_b, edge_fc_w, edge_fc_b, inv_w, inv_b, mix_w, mix_b)
    table_t = tab_t.reshape(-1)
    parts, att = _sc_edges(table_t, jnp.asarray(perm_pad),
                           edge_index[0], edge_index[1], jnp.asarray(n1),
                           init0)
    acc2d = _tc_merge(parts)
    accr = acc2d.reshape(NP, 4)[:N]
    perm_pred = accr[:, 2:4]
    xc_pred = accr[:, 0:2]
    edge_att = att.reshape(E, 1)
    return (perm_pred, xc_pred, feature_mask, edge_att)


# speedup vs baseline: 46.3821x; 1.0798x over previous
import functools

import jax
import jax.numpy as jnp
import numpy as np
from jax import lax
from jax.experimental import pallas as pl
from jax.experimental.pallas import tpu as pltpu
from jax.experimental.pallas import tpu_sc as plsc

N = 10000
E = 320000
DIM_IN = 128

NS = 16
NP = 10240
NPT = NP // NS
NP4 = NP * 4
EPW = E // (2 * NS)
C = 2000
L = 16
NR = NP4 // 128
GR = 80
RPT = NR // NS


def _raw_consts():
    nkey = jax.random.key(42)

    def logit_noise(key, shape):
        u = jax.random.uniform(key, shape, minval=1e-10, maxval=1.0 - 1e-10,
                               dtype=jnp.float32)
        return jnp.log(u) - jnp.log(1.0 - u)

    n0 = logit_noise(jax.random.fold_in(nkey, 0), (N, DIM_IN - 2))
    n1 = logit_noise(jax.random.fold_in(nkey, 1), (E, 1))
    perm = jax.random.permutation(jax.random.fold_in(nkey, 2), N)
    return n0, n1, perm


def _consts_np():
    cpu = jax.devices("cpu")[0]
    with jax.default_device(cpu):
        n0, n1, perm = _raw_consts()
        n0, n1, perm = np.asarray(n0), np.asarray(n1), np.asarray(perm)
    n0_pad = np.zeros((NP, DIM_IN - 2), dtype=np.float32)
    n0_pad[:N] = n0
    perm_pad = np.zeros((NP,), dtype=np.int32)
    perm_pad[:N] = perm.astype(np.int32)
    return n0_pad, n1.reshape(E).astype(np.float32), perm_pad


try:
    _CONSTS = _consts_np()
except Exception:
    _CONSTS = None


def _get_consts():
    if _CONSTS is not None:
        return _CONSTS
    n0, n1, perm = _raw_consts()
    n0_pad = jnp.zeros((NP, DIM_IN - 2), jnp.float32).at[:N].set(n0)
    perm_pad = jnp.zeros((NP,), jnp.int32).at[:N].set(perm.astype(jnp.int32))
    return n0_pad, n1.reshape(E), perm_pad

_BN = 2048


def _tc_body(x_ref, nz_ref, wf_ref, bf_ref, gw_ref, gb_ref, gow_ref,
             gob_ref, efw_ref, eb_ref, ivw_ref, ivb_ref, mxw_ref, mxb_ref,
             mask_ref, tab_ref, init_ref):
    f32 = jnp.float32
    xb = x_ref[...]
    ml = jnp.dot(xb, wf_ref[...], preferred_element_type=f32)
    mask = jax.nn.sigmoid(ml + bf_ref[...] + nz_ref[...])
    mask_ref[...] = mask
    x126 = xb[:, : DIM_IN - 2]
    xl = xb[:, DIM_IN - 2:]
    y = jnp.concatenate([x126 * mask, xl], axis=1)

    ivw = ivw_ref[...]
    mxw = mxw_ref[...]
    a = jnp.dot(gw_ref[...], gow_ref[...], preferred_element_type=f32)
    wc = jnp.dot(a, ivw, preferred_element_type=f32)
    wp = jnp.dot(a, mxw, preferred_element_type=f32)
    row = lax.broadcasted_iota(jnp.int32, wp.shape, 0)
    wp_z = jnp.where(row < DIM_IN - 2, wp, 0.0)
    gba = jnp.dot(gb_ref[...], gow_ref[...], preferred_element_type=f32)
    bc = jnp.dot(gba, ivw, preferred_element_type=f32)
    bp = jnp.dot(gba, mxw, preferred_element_type=f32)
    efw = efw_ref[...]
    ews = efw[: DIM_IN]
    ewd = efw[DIM_IN:]

    uc = jnp.dot(y, wc, preferred_element_type=f32) + bc
    upb = jnp.dot(y, wp, preferred_element_type=f32) + bp
    v = jnp.dot(xb - y, wp_z, preferred_element_type=f32)
    es = jnp.dot(xb, ews, preferred_element_type=f32) + eb_ref[...]
    ed = jnp.dot(xb, ewd, preferred_element_type=f32)
    tab = jnp.concatenate([uc, upb, v, es, ed], axis=1)
    tab_ref[...] = tab.T

    gob = gob_ref[...]
    ccc = jnp.dot(gob, ivw, preferred_element_type=f32) + ivb_ref[...]
    ccp = jnp.dot(gob, mxw, preferred_element_type=f32) + mxb_ref[...]
    cv4 = jnp.concatenate([ccc, ccp], axis=1)
    init_ref[...] = jnp.tile(cv4, (NR, 32))


def _tc_dense(x, nz_pad, feature_fc_w, feature_fc_b, gat_w, gat_b,
              gat_out_w, gat_out_b, edge_fc_w, edge_fc_b,
              inv_w, inv_b, mix_w, mix_b, interpret=False):
    grid = (NP // _BN,)
    d2 = DIM_IN - 2
    full = lambda shape: pl.BlockSpec(shape, lambda i: tuple(0 for _ in shape))
    return pl.pallas_call(
        _tc_body,
        grid=grid,
        in_specs=[
            pl.BlockSpec((_BN, DIM_IN), lambda i: (i, 0)),
            pl.BlockSpec((_BN, d2), lambda i: (i, 0)),
            full((DIM_IN, d2)),
            full((1, d2)),
            full((DIM_IN, DIM_IN)),
            full((1, DIM_IN)),
            full((DIM_IN, 2)),
            full((1, 2)),
            full((2 * DIM_IN, 1)),
            full((1, 1)),
            full((2, 2)),
            full((1, 2)),
            full((2, 2)),
            full((1, 2)),
        ],
        out_specs=[
            pl.BlockSpec((_BN, d2), lambda i: (i, 0)),
            pl.BlockSpec((8, _BN), lambda i: (0, i)),
            pl.BlockSpec((NR, 128), lambda i: (0, 0)),
        ],
        out_shape=[
            jax.ShapeDtypeStruct((N, d2), jnp.float32),
            jax.ShapeDtypeStruct((8, NP), jnp.float32),
            jax.ShapeDtypeStruct((NR, 128), jnp.float32),
        ],
        interpret=interpret,
    )(x, nz_pad, feature_fc_w, feature_fc_b.reshape(1, d2),
      gat_w, gat_b.reshape(1, DIM_IN), gat_out_w, gat_out_b.reshape(1, 2),
      edge_fc_w, edge_fc_b.reshape(1, 1), inv_w, inv_b.reshape(1, 2),
      mix_w, mix_b.reshape(1, 2))


def _sc_body(tab_ref, perm_ref, src_ref, dst_ref, nzE_ref, init0_ref,
             acc_out, att_out,
             t0, t1, t2, t3, t4, t5, acc, srcb, dstb, nzb, attb,
             srcb1, dstb1, nzb1, attb1, permb,
             tmpacc, idxb, sem, sem2, sh_up0, sh_up1, sh_acc):
    cid = lax.axis_index("c")
    wid = lax.axis_index("s")
    ew = cid * NS + wid
    nbase = pl.multiple_of(wid * NPT, 8)

    cps = [
        pltpu.async_copy(tab_ref.at[pl.ds(2 * NP, NP)], t0, sem),
        pltpu.async_copy(tab_ref.at[pl.ds(3 * NP, NP)], t1, sem),
        pltpu.async_copy(tab_ref.at[pl.ds(4 * NP, NP)], t2, sem),
        pltpu.async_copy(tab_ref.at[pl.ds(5 * NP, NP)], t3, sem),
        pltpu.async_copy(tab_ref.at[pl.ds(6 * NP, NP)], t4, sem),
        pltpu.async_copy(tab_ref.at[pl.ds(7 * NP, NP)], t5, sem),
        pltpu.async_copy(perm_ref.at[pl.ds(nbase, NPT)], permb, sem),
    ]

    @pl.when(ew == 0)
    def _():
        pltpu.sync_copy(init0_ref, acc)

    @pl.when(ew != 0)
    def _():
        zv = jnp.zeros((L,), jnp.float32)

        @plsc.parallel_loop(0, NR, 1, unroll=2)
        def _(i):
            for jj in range(128 // L):
                acc[i, pl.ds(jj * L, L)] = zv

    iota16 = lax.iota(jnp.int32, L)
    for g in range(NR // L):
        idxb[pl.ds(g * L, L)] = iota16 + L * g

    for cp in cps:
        cp.wait()

    @plsc.parallel_loop(0, NPT // L, 1, unroll=2)
    def _(j):
        p = permb[pl.ds(j * L, L)]
        g0 = plsc.load_gather(t2, [p])
        g1 = plsc.load_gather(t3, [p])
        o = nbase + j * L
        t0[pl.ds(o, L)] = t0[pl.ds(o, L)] + g0
        t1[pl.ds(o, L)] = t1[pl.ds(o, L)] + g1
    pltpu.sync_copy(t0.at[pl.ds(nbase, NPT)], sh_up0.at[pl.ds(nbase, NPT)])
    pltpu.sync_copy(t1.at[pl.ds(nbase, NPT)], sh_up1.at[pl.ds(nbase, NPT)])
    pltpu.sync_copy(tab_ref.at[pl.ds(0, NP)], t2)
    pltpu.sync_copy(tab_ref.at[pl.ds(NP, NP)], t3)
    plsc.subcore_barrier()
    pltpu.sync_copy(sh_up0, t0)
    pltpu.sync_copy(sh_up1, t1)

    nch = EPW // C

    def _eb(ch):
        return pl.multiple_of(ew * EPW + ch * C, 8)

    bufs = [(srcb, dstb, nzb, attb), (srcb1, dstb1, nzb1, attb1)]

    def _issue_loads(ch, par):
        eb = _eb(ch)
        sb, db, nb, _ = bufs[par]
        return [
            pltpu.async_copy(src_ref.at[pl.ds(eb, C)], sb, sem),
            pltpu.async_copy(dst_ref.at[pl.ds(eb, C)], db, sem),
            pltpu.async_copy(nzE_ref.at[pl.ds(eb, C)], nb, sem),
        ]

    pend = _issue_loads(0, 0)
    st = []
    for ch in range(nch):
        par = ch & 1
        for cp in pend:
            cp.wait()
        if ch + 1 < nch:
            pend = _issue_loads(ch + 1, 1 - par)
        for cp in st:
            cp.wait()
        st = []
        sb, db, nb, ab = bufs[par]

        @plsc.parallel_loop(0, C // L, 1, unroll=8)
        def _(j):
            s = sb[pl.ds(j * L, L)]
            d = db[pl.ds(j * L, L)]
            esv = plsc.load_gather(t4, [s])
            edv = plsc.load_gather(t5, [d])
            lg = esv + edv + nb[pl.ds(j * L, L)]
            att = 1.0 / (1.0 + jnp.exp(-lg))
            ab[pl.ds(j * L, L)] = att
            uc0 = plsc.load_gather(t2, [s])
            uc1 = plsc.load_gather(t3, [s])
            up0 = plsc.load_gather(t0, [s])
            up1 = plsc.load_gather(t1, [s])
            r = lax.shift_right_logical(d, 5)
            lb = lax.shift_left(d & 31, 2)
            plsc.addupdate_scatter(acc, [r, lb], att * uc0)
            plsc.addupdate_scatter(acc, [r, lb + 1], att * uc1)
            plsc.addupdate_scatter(acc, [r, lb + 2], att * up0)
            plsc.addupdate_scatter(acc, [r, lb + 3], att * up1)

        st = [pltpu.async_copy(ab, att_out.at[pl.ds(_eb(ch), C)], sem2)]
    for cp in st:
        cp.wait()

    @pl.when(wid == 0)
    def _():
        pltpu.sync_copy(acc, sh_acc)

    plsc.subcore_barrier()

    @pl.when(wid != 0)
    def _():
        pltpu.sync_copy(acc, sh_acc.at[idxb], add=True)

    plsc.subcore_barrier()

    @pl.when(wid < 8)
    def _():
        ob = pl.multiple_of(wid * 40, 8)
        pltpu.sync_copy(sh_acc.at[pl.ds(ob, 40)], tmpacc)
        pltpu.sync_copy(tmpacc, acc_out.at[cid, pl.ds(ob, 40)])


def _sc_edges(table_t, perm_pad, src, dst, nz, init0):
    mesh = plsc.VectorSubcoreMesh(core_axis_name="c", subcore_axis_name="s",
                                  num_cores=2)
    f32 = jnp.float32
    i32 = jnp.int32
    kfn = pl.kernel(
        _sc_body,
        compiler_params=pltpu.CompilerParams(needs_layout_passes=False),
        out_type=[
            jax.ShapeDtypeStruct((2, NR, 128), f32),
            jax.ShapeDtypeStruct((E,), f32),
        ],
        mesh=mesh,
        scratch_types=[
            pltpu.VMEM((NP,), f32),
            pltpu.VMEM((NP,), f32),
            pltpu.VMEM((NP,), f32),
            pltpu.VMEM((NP,), f32),
            pltpu.VMEM((NP,), f32),
            pltpu.VMEM((NP,), f32),
            pltpu.VMEM((NR, 128), f32),
            pltpu.VMEM((C,), i32),
            pltpu.VMEM((C,), i32),
            pltpu.VMEM((C,), f32),
            pltpu.VMEM((C,), f32),
            pltpu.VMEM((C,), i32),
            pltpu.VMEM((C,), i32),
            pltpu.VMEM((C,), f32),
            pltpu.VMEM((C,), f32),
            pltpu.VMEM((NPT,), i32),
            pltpu.VMEM((40, 128), f32),
            pltpu.VMEM((NR,), i32),
            pltpu.SemaphoreType.DMA,
            pltpu.SemaphoreType.DMA,
            pltpu.VMEM_SHARED((NP,), f32),
            pltpu.VMEM_SHARED((NP,), f32),
            pltpu.VMEM_SHARED((NR, 128), f32),
        ],
    )
    return kfn(table_t, perm_pad, src, dst, nz, init0)


def _tc_merge_body(p_ref, out_ref):
    out_ref[...] = p_ref[0] + p_ref[1]


def _tc_merge(parts):
    return pl.pallas_call(
        _tc_merge_body,
        out_shape=jax.ShapeDtypeStruct((NR, 128), jnp.float32),
    )(parts)


def kernel(x, edge_index, tg_mask, batch, ptr,
           feature_fc_w, feature_fc_b, edge_fc_w, edge_fc_b,
           gat_w, gat_b, gat_out_w, gat_out_b,
           inv_w, inv_b, mix_w, mix_b):
    n0_pad, n1, perm_pad = _get_consts()
    f32 = jnp.float32

    feature_mask, tab_t, init0 = _tc_dense(
        x, jnp.asarray(n0_pad), feature_fc_w, feature_fc_b, gat_w, gat_b,
        gat_out_w, gat_out_b, edge_fc_w, edge_fc_b, inv_w, inv_b, mix_w, mix_b)
    table_t = tab_t.reshape(-1)
    parts, att = _sc_edges(table_t, jnp.asarray(perm_pad),
                           edge_index[0], edge_index[1], jnp.asarray(n1),
                           init0)
    acc2d = _tc_merge(parts)
    accr = acc2d.reshape(NP, 4)[:N]
    perm_pred = accr[:, 2:4]
    xc_pred = accr[:, 0:2]
    edge_att = att.reshape(E, 1)
    return (perm_pred, xc_pred, feature_mask, edge_att)


# speedup vs baseline: 52.7498x; 1.1373x over previous
import functools

import jax
import jax.numpy as jnp
import numpy as np
from jax import lax
from jax.experimental import pallas as pl
from jax.experimental.pallas import tpu as pltpu
from jax.experimental.pallas import tpu_sc as plsc

N = 10000
E = 320000
DIM_IN = 128

NS = 16
NP = 10240
NPT = NP // NS
NP4 = NP * 4
EPW = E // (2 * NS)
C = 2000
L = 16
NR = NP4 // 128
GR = 80
RPT = NR // NS


def _raw_consts():
    nkey = jax.random.key(42)

    def logit_noise(key, shape):
        u = jax.random.uniform(key, shape, minval=1e-10, maxval=1.0 - 1e-10,
                               dtype=jnp.float32)
        return jnp.log(u) - jnp.log(1.0 - u)

    n0 = logit_noise(jax.random.fold_in(nkey, 0), (N, DIM_IN - 2))
    n1 = logit_noise(jax.random.fold_in(nkey, 1), (E, 1))
    perm = jax.random.permutation(jax.random.fold_in(nkey, 2), N)
    return n0, n1, perm


def _consts_np():
    cpu = jax.devices("cpu")[0]
    with jax.default_device(cpu):
        n0, n1, perm = _raw_consts()
        n0, n1, perm = np.asarray(n0), np.asarray(n1), np.asarray(perm)
    n0_pad = np.zeros((NP, DIM_IN - 2), dtype=np.float32)
    n0_pad[:N] = n0
    perm_pad = np.zeros((NP,), dtype=np.int32)
    perm_pad[:N] = perm.astype(np.int32)
    return n0_pad, n1.reshape(E).astype(np.float32), perm_pad


try:
    _CONSTS = _consts_np()
except Exception:
    _CONSTS = None


def _get_consts():
    if _CONSTS is not None:
        return _CONSTS
    n0, n1, perm = _raw_consts()
    n0_pad = jnp.zeros((NP, DIM_IN - 2), jnp.float32).at[:N].set(n0)
    perm_pad = jnp.zeros((NP,), jnp.int32).at[:N].set(perm.astype(jnp.int32))
    return n0_pad, n1.reshape(E), perm_pad

_BN = 2048


def _tc_body(x_ref, nz_ref, wf_ref, bf_ref, gw_ref, gb_ref, gow_ref,
             gob_ref, efw_ref, eb_ref, ivw_ref, ivb_ref, mxw_ref, mxb_ref,
             ei_ref,
             mask_ref, tab_ref, init_ref, src_ref, dst_ref):
    f32 = jnp.float32
    ei = ei_ref[...]
    src_ref[...] = ei[0]
    dst_ref[...] = ei[1]
    xb = x_ref[...]
    ml = jnp.dot(xb, wf_ref[...], preferred_element_type=f32)
    mask = jax.nn.sigmoid(ml + bf_ref[...] + nz_ref[...])
    mask_ref[...] = mask
    x126 = xb[:, : DIM_IN - 2]
    xl = xb[:, DIM_IN - 2:]
    y = jnp.concatenate([x126 * mask, xl], axis=1)

    ivw = ivw_ref[...]
    mxw = mxw_ref[...]
    a = jnp.dot(gw_ref[...], gow_ref[...], preferred_element_type=f32)
    wc = jnp.dot(a, ivw, preferred_element_type=f32)
    wp = jnp.dot(a, mxw, preferred_element_type=f32)
    row = lax.broadcasted_iota(jnp.int32, wp.shape, 0)
    wp_z = jnp.where(row < DIM_IN - 2, wp, 0.0)
    gba = jnp.dot(gb_ref[...], gow_ref[...], preferred_element_type=f32)
    bc = jnp.dot(gba, ivw, preferred_element_type=f32)
    bp = jnp.dot(gba, mxw, preferred_element_type=f32)
    efw = efw_ref[...]
    ews = efw[: DIM_IN]
    ewd = efw[DIM_IN:]

    uc = jnp.dot(y, wc, preferred_element_type=f32) + bc
    upb = jnp.dot(y, wp, preferred_element_type=f32) + bp
    v = jnp.dot(xb - y, wp_z, preferred_element_type=f32)
    es = jnp.dot(xb, ews, preferred_element_type=f32) + eb_ref[...]
    ed = jnp.dot(xb, ewd, preferred_element_type=f32)
    tab = jnp.concatenate([uc, upb, v, es, ed], axis=1)
    tab_ref[...] = tab.T

    gob = gob_ref[...]
    ccc = jnp.dot(gob, ivw, preferred_element_type=f32) + ivb_ref[...]
    ccp = jnp.dot(gob, mxw, preferred_element_type=f32) + mxb_ref[...]
    cv4 = jnp.concatenate([ccc, ccp], axis=1)
    init_ref[...] = jnp.tile(cv4, (NR, 32))


def _tc_dense(x, nz_pad, feature_fc_w, feature_fc_b, gat_w, gat_b,
              gat_out_w, gat_out_b, edge_fc_w, edge_fc_b,
              inv_w, inv_b, mix_w, mix_b, edge_index, interpret=False):
    grid = (NP // _BN,)
    eb_blk = 65536
    d2 = DIM_IN - 2
    full = lambda shape: pl.BlockSpec(shape, lambda i: tuple(0 for _ in shape))
    return pl.pallas_call(
        _tc_body,
        grid=grid,
        in_specs=[
            pl.BlockSpec((_BN, DIM_IN), lambda i: (i, 0)),
            pl.BlockSpec((_BN, d2), lambda i: (i, 0)),
            full((DIM_IN, d2)),
            full((1, d2)),
            full((DIM_IN, DIM_IN)),
            full((1, DIM_IN)),
            full((DIM_IN, 2)),
            full((1, 2)),
            full((2 * DIM_IN, 1)),
            full((1, 1)),
            full((2, 2)),
            full((1, 2)),
            full((2, 2)),
            full((1, 2)),
            pl.BlockSpec((2, eb_blk), lambda i: (0, i)),
        ],
        out_specs=[
            pl.BlockSpec((_BN, d2), lambda i: (i, 0)),
            pl.BlockSpec((8, _BN), lambda i: (0, i)),
            pl.BlockSpec((NR, 128), lambda i: (0, 0)),
            pl.BlockSpec((eb_blk,), lambda i: (i,)),
            pl.BlockSpec((eb_blk,), lambda i: (i,)),
        ],
        out_shape=[
            jax.ShapeDtypeStruct((N, d2), jnp.float32),
            jax.ShapeDtypeStruct((8, NP), jnp.float32),
            jax.ShapeDtypeStruct((NR, 128), jnp.float32),
            jax.ShapeDtypeStruct((E,), jnp.int32),
            jax.ShapeDtypeStruct((E,), jnp.int32),
        ],
        interpret=interpret,
    )(x, nz_pad, feature_fc_w, feature_fc_b.reshape(1, d2),
      gat_w, gat_b.reshape(1, DIM_IN), gat_out_w, gat_out_b.reshape(1, 2),
      edge_fc_w, edge_fc_b.reshape(1, 1), inv_w, inv_b.reshape(1, 2),
      mix_w, mix_b.reshape(1, 2), edge_index)


def _sc_body(tab_ref, perm_ref, src_ref, dst_ref, nzE_ref, init0_ref,
             acc_out, att_out,
             t0, t1, t2, t3, t4, t5, acc, srcb, dstb, nzb, attb,
             srcb1, dstb1, nzb1, attb1, permb,
             tmpacc, idxb, sem, sem2, sh_up0, sh_up1, sh_acc):
    cid = lax.axis_index("c")
    wid = lax.axis_index("s")
    ew = cid * NS + wid
    nbase = pl.multiple_of(wid * NPT, 8)

    cps = [
        pltpu.async_copy(tab_ref.at[pl.ds(2 * NP, NP)], t0, sem),
        pltpu.async_copy(tab_ref.at[pl.ds(3 * NP, NP)], t1, sem),
        pltpu.async_copy(tab_ref.at[pl.ds(4 * NP, NP)], t2, sem),
        pltpu.async_copy(tab_ref.at[pl.ds(5 * NP, NP)], t3, sem),
        pltpu.async_copy(tab_ref.at[pl.ds(6 * NP, NP)], t4, sem),
        pltpu.async_copy(tab_ref.at[pl.ds(7 * NP, NP)], t5, sem),
        pltpu.async_copy(perm_ref.at[pl.ds(nbase, NPT)], permb, sem),
    ]

    @pl.when(ew == 0)
    def _():
        pltpu.sync_copy(init0_ref, acc)

    @pl.when(ew != 0)
    def _():
        zv = jnp.zeros((L,), jnp.float32)

        @plsc.parallel_loop(0, NR, 1, unroll=2)
        def _(i):
            for jj in range(128 // L):
                acc[i, pl.ds(jj * L, L)] = zv

    iota16 = lax.iota(jnp.int32, L)
    for g in range(NR // L):
        idxb[pl.ds(g * L, L)] = iota16 + L * g

    for cp in cps:
        cp.wait()

    @plsc.parallel_loop(0, NPT // L, 1, unroll=2)
    def _(j):
        p = permb[pl.ds(j * L, L)]
        g0 = plsc.load_gather(t2, [p])
        g1 = plsc.load_gather(t3, [p])
        o = nbase + j * L
        t0[pl.ds(o, L)] = t0[pl.ds(o, L)] + g0
        t1[pl.ds(o, L)] = t1[pl.ds(o, L)] + g1
    pltpu.sync_copy(t0.at[pl.ds(nbase, NPT)], sh_up0.at[pl.ds(nbase, NPT)])
    pltpu.sync_copy(t1.at[pl.ds(nbase, NPT)], sh_up1.at[pl.ds(nbase, NPT)])
    pltpu.sync_copy(tab_ref.at[pl.ds(0, NP)], t2)
    pltpu.sync_copy(tab_ref.at[pl.ds(NP, NP)], t3)
    plsc.subcore_barrier()
    pltpu.sync_copy(sh_up0, t0)
    pltpu.sync_copy(sh_up1, t1)

    nch = EPW // C

    def _eb(ch):
        return pl.multiple_of(ew * EPW + ch * C, 8)

    bufs = [(srcb, dstb, nzb, attb), (srcb1, dstb1, nzb1, attb1)]

    def _issue_loads(ch, par):
        eb = _eb(ch)
        sb, db, nb, _ = bufs[par]
        return [
            pltpu.async_copy(src_ref.at[pl.ds(eb, C)], sb, sem),
            pltpu.async_copy(dst_ref.at[pl.ds(eb, C)], db, sem),
            pltpu.async_copy(nzE_ref.at[pl.ds(eb, C)], nb, sem),
        ]

    pend = _issue_loads(0, 0)
    st = []
    for ch in range(nch):
        par = ch & 1
        for cp in pend:
            cp.wait()
        if ch + 1 < nch:
            pend = _issue_loads(ch + 1, 1 - par)
        for cp in st:
            cp.wait()
        st = []
        sb, db, nb, ab = bufs[par]

        @plsc.parallel_loop(0, C // L, 1, unroll=8)
        def _(j):
            s = sb[pl.ds(j * L, L)]
            d = db[pl.ds(j * L, L)]
            esv = plsc.load_gather(t4, [s])
            edv = plsc.load_gather(t5, [d])
            lg = esv + edv + nb[pl.ds(j * L, L)]
            att = 1.0 / (1.0 + jnp.exp(-lg))
            ab[pl.ds(j * L, L)] = att
            uc0 = plsc.load_gather(t2, [s])
            uc1 = plsc.load_gather(t3, [s])
            up0 = plsc.load_gather(t0, [s])
            up1 = plsc.load_gather(t1, [s])
            r = lax.shift_right_logical(d, 5)
            lb = lax.shift_left(d & 31, 2)
            plsc.addupdate_scatter(acc, [r, lb], att * uc0)
            plsc.addupdate_scatter(acc, [r, lb + 1], att * uc1)
            plsc.addupdate_scatter(acc, [r, lb + 2], att * up0)
            plsc.addupdate_scatter(acc, [r, lb + 3], att * up1)

        st = [pltpu.async_copy(ab, att_out.at[pl.ds(_eb(ch), C)], sem2)]
    for cp in st:
        cp.wait()

    @pl.when(wid == 0)
    def _():
        pltpu.sync_copy(acc, sh_acc)

    plsc.subcore_barrier()

    @pl.when(wid != 0)
    def _():
        pltpu.sync_copy(acc, sh_acc.at[idxb], add=True)

    plsc.subcore_barrier()

    @pl.when(wid < 8)
    def _():
        ob = pl.multiple_of(wid * 40, 8)
        pltpu.sync_copy(sh_acc.at[pl.ds(ob, 40)], tmpacc)
        pltpu.sync_copy(tmpacc, acc_out.at[cid, pl.ds(ob, 40)])


def _sc_edges(table_t, perm_pad, src, dst, nz, init0):
    mesh = plsc.VectorSubcoreMesh(core_axis_name="c", subcore_axis_name="s",
                                  num_cores=2)
    f32 = jnp.float32
    i32 = jnp.int32
    kfn = pl.kernel(
        _sc_body,
        compiler_params=pltpu.CompilerParams(needs_layout_passes=False),
        out_type=[
            jax.ShapeDtypeStruct((2, NR, 128), f32),
            jax.ShapeDtypeStruct((E,), f32),
        ],
        mesh=mesh,
        scratch_types=[
            pltpu.VMEM((NP,), f32),
            pltpu.VMEM((NP,), f32),
            pltpu.VMEM((NP,), f32),
            pltpu.VMEM((NP,), f32),
            pltpu.VMEM((NP,), f32),
            pltpu.VMEM((NP,), f32),
            pltpu.VMEM((NR, 128), f32),
            pltpu.VMEM((C,), i32),
            pltpu.VMEM((C,), i32),
            pltpu.VMEM((C,), f32),
            pltpu.VMEM((C,), f32),
            pltpu.VMEM((C,), i32),
            pltpu.VMEM((C,), i32),
            pltpu.VMEM((C,), f32),
            pltpu.VMEM((C,), f32),
            pltpu.VMEM((NPT,), i32),
            pltpu.VMEM((40, 128), f32),
            pltpu.VMEM((NR,), i32),
            pltpu.SemaphoreType.DMA,
            pltpu.SemaphoreType.DMA,
            pltpu.VMEM_SHARED((NP,), f32),
            pltpu.VMEM_SHARED((NP,), f32),
            pltpu.VMEM_SHARED((NR, 128), f32),
        ],
    )
    return kfn(table_t, perm_pad, src, dst, nz, init0)


def _tc_merge_body(p_ref, out_ref):
    out_ref[...] = p_ref[0] + p_ref[1]


def _tc_merge(parts):
    return pl.pallas_call(
        _tc_merge_body,
        out_shape=jax.ShapeDtypeStruct((NR, 128), jnp.float32),
    )(parts)


def kernel(x, edge_index, tg_mask, batch, ptr,
           feature_fc_w, feature_fc_b, edge_fc_w, edge_fc_b,
           gat_w, gat_b, gat_out_w, gat_out_b,
           inv_w, inv_b, mix_w, mix_b):
    n0_pad, n1, perm_pad = _get_consts()
    f32 = jnp.float32

    feature_mask, tab_t, init0, e_src, e_dst = _tc_dense(
        x, jnp.asarray(n0_pad), feature_fc_w, feature_fc_b, gat_w, gat_b,
        gat_out_w, gat_out_b, edge_fc_w, edge_fc_b, inv_w, inv_b, mix_w, mix_b,
        edge_index)
    table_t = tab_t.reshape(-1)
    parts, att = _sc_edges(table_t, jnp.asarray(perm_pad),
                           e_src, e_dst, jnp.asarray(n1), init0)
    acc2d = _tc_merge(parts)
    accr = acc2d.reshape(NP, 4)[:N]
    perm_pred = accr[:, 2:4]
    xc_pred = accr[:, 0:2]
    edge_att = att.reshape(E, 1)
    return (perm_pred, xc_pred, feature_mask, edge_att)


# speedup vs baseline: 53.2809x; 1.0101x over previous
import functools

import jax
import jax.numpy as jnp
import numpy as np
from jax import lax
from jax.experimental import pallas as pl
from jax.experimental.pallas import tpu as pltpu
from jax.experimental.pallas import tpu_sc as plsc

N = 10000
E = 320000
DIM_IN = 128

NS = 16
NP = 10240
NPT = NP // NS
NP4 = NP * 4
EPW = E // (2 * NS)
C = 2000
L = 16
NR = NP4 // 128
GR = 80
RPT = NR // NS


def _raw_consts():
    nkey = jax.random.key(42)

    def logit_noise(key, shape):
        u = jax.random.uniform(key, shape, minval=1e-10, maxval=1.0 - 1e-10,
                               dtype=jnp.float32)
        return jnp.log(u) - jnp.log(1.0 - u)

    n0 = logit_noise(jax.random.fold_in(nkey, 0), (N, DIM_IN - 2))
    n1 = logit_noise(jax.random.fold_in(nkey, 1), (E, 1))
    perm = jax.random.permutation(jax.random.fold_in(nkey, 2), N)
    return n0, n1, perm


def _consts_np():
    cpu = jax.devices("cpu")[0]
    with jax.default_device(cpu):
        n0, n1, perm = _raw_consts()
        n0, n1, perm = np.asarray(n0), np.asarray(n1), np.asarray(perm)
    n0_pad = np.zeros((NP, DIM_IN - 2), dtype=np.float32)
    n0_pad[:N] = n0
    perm_pad = np.zeros((NP,), dtype=np.int32)
    perm_pad[:N] = perm.astype(np.int32)
    return n0_pad, n1.reshape(E).astype(np.float32), perm_pad


try:
    _CONSTS = _consts_np()
except Exception:
    _CONSTS = None


def _get_consts():
    if _CONSTS is not None:
        return _CONSTS
    n0, n1, perm = _raw_consts()
    n0_pad = jnp.zeros((NP, DIM_IN - 2), jnp.float32).at[:N].set(n0)
    perm_pad = jnp.zeros((NP,), jnp.int32).at[:N].set(perm.astype(jnp.int32))
    return n0_pad, n1.reshape(E), perm_pad

_BN = 2048


def _tc_body(x_ref, nz_ref, wf_ref, bf_ref, gw_ref, gb_ref, gow_ref,
             gob_ref, efw_ref, eb_ref, ivw_ref, ivb_ref, mxw_ref, mxb_ref,
             ei_ref,
             mask_ref, tab_ref, init_ref, src_ref, dst_ref):
    f32 = jnp.float32
    ei = ei_ref[...]
    src_ref[...] = ei[0]
    dst_ref[...] = ei[1]
    xb = x_ref[...]
    ml = jnp.dot(xb, wf_ref[...], preferred_element_type=f32)
    mask = jax.nn.sigmoid(ml + bf_ref[...] + nz_ref[...])
    mask_ref[...] = mask
    x126 = xb[:, : DIM_IN - 2]
    xl = xb[:, DIM_IN - 2:]
    y = jnp.concatenate([x126 * mask, xl], axis=1)

    ivw = ivw_ref[...]
    mxw = mxw_ref[...]
    a = jnp.dot(gw_ref[...], gow_ref[...], preferred_element_type=f32)
    wc = jnp.dot(a, ivw, preferred_element_type=f32)
    wp = jnp.dot(a, mxw, preferred_element_type=f32)
    row = lax.broadcasted_iota(jnp.int32, wp.shape, 0)
    wp_z = jnp.where(row < DIM_IN - 2, wp, 0.0)
    gba = jnp.dot(gb_ref[...], gow_ref[...], preferred_element_type=f32)
    bc = jnp.dot(gba, ivw, preferred_element_type=f32)
    bp = jnp.dot(gba, mxw, preferred_element_type=f32)
    efw = efw_ref[...]
    ews = efw[: DIM_IN]
    ewd = efw[DIM_IN:]

    uc = jnp.dot(y, wc, preferred_element_type=f32) + bc
    upb = jnp.dot(y, wp, preferred_element_type=f32) + bp
    v = jnp.dot(xb - y, wp_z, preferred_element_type=f32)
    es = jnp.dot(xb, ews, preferred_element_type=f32) + eb_ref[...]
    ed = jnp.dot(xb, ewd, preferred_element_type=f32)
    tab = jnp.concatenate([uc, upb, v, es, ed], axis=1)
    tab_ref[...] = tab.T

    gob = gob_ref[...]
    ccc = jnp.dot(gob, ivw, preferred_element_type=f32) + ivb_ref[...]
    ccp = jnp.dot(gob, mxw, preferred_element_type=f32) + mxb_ref[...]
    cv4 = jnp.concatenate([ccc, ccp], axis=1)
    init_ref[...] = jnp.tile(cv4, (NR, 32))


def _tc_dense(x, nz_pad, feature_fc_w, feature_fc_b, gat_w, gat_b,
              gat_out_w, gat_out_b, edge_fc_w, edge_fc_b,
              inv_w, inv_b, mix_w, mix_b, edge_index, interpret=False):
    grid = (NP // _BN,)
    eb_blk = 65536
    d2 = DIM_IN - 2
    full = lambda shape: pl.BlockSpec(shape, lambda i: tuple(0 for _ in shape))
    return pl.pallas_call(
        _tc_body,
        grid=grid,
        in_specs=[
            pl.BlockSpec((_BN, DIM_IN), lambda i: (i, 0)),
            pl.BlockSpec((_BN, d2), lambda i: (i, 0)),
            full((DIM_IN, d2)),
            full((1, d2)),
            full((DIM_IN, DIM_IN)),
            full((1, DIM_IN)),
            full((DIM_IN, 2)),
            full((1, 2)),
            full((2 * DIM_IN, 1)),
            full((1, 1)),
            full((2, 2)),
            full((1, 2)),
            full((2, 2)),
            full((1, 2)),
            pl.BlockSpec((2, eb_blk), lambda i: (0, i)),
        ],
        out_specs=[
            pl.BlockSpec((_BN, d2), lambda i: (i, 0)),
            pl.BlockSpec((8, _BN), lambda i: (0, i)),
            pl.BlockSpec((NR, 128), lambda i: (0, 0)),
            pl.BlockSpec((eb_blk,), lambda i: (i,)),
            pl.BlockSpec((eb_blk,), lambda i: (i,)),
        ],
        out_shape=[
            jax.ShapeDtypeStruct((N, d2), jnp.float32),
            jax.ShapeDtypeStruct((8, NP), jnp.float32),
            jax.ShapeDtypeStruct((NR, 128), jnp.float32),
            jax.ShapeDtypeStruct((E,), jnp.int32),
            jax.ShapeDtypeStruct((E,), jnp.int32),
        ],
        interpret=interpret,
    )(x, nz_pad, feature_fc_w, feature_fc_b.reshape(1, d2),
      gat_w, gat_b.reshape(1, DIM_IN), gat_out_w, gat_out_b.reshape(1, 2),
      edge_fc_w, edge_fc_b.reshape(1, 1), inv_w, inv_b.reshape(1, 2),
      mix_w, mix_b.reshape(1, 2), edge_index)


def _sc_body(tab_ref, perm_ref, src_ref, dst_ref, nzE_ref, init0_ref,
             acc_out, att_out,
             t0, t1, t2, t3, t4, t5, acc, srcb, dstb, nzb, attb,
             srcb1, dstb1, nzb1, attb1, permb,
             tmpacc, idxb, sem, sem2, sem3, sh_up0, sh_up1, sh_acc):
    cid = lax.axis_index("c")
    wid = lax.axis_index("s")
    ew = cid * NS + wid
    nbase = pl.multiple_of(wid * NPT, 8)

    cps = [
        pltpu.async_copy(tab_ref.at[pl.ds(2 * NP, NP)], t0, sem),
        pltpu.async_copy(tab_ref.at[pl.ds(3 * NP, NP)], t1, sem),
        pltpu.async_copy(tab_ref.at[pl.ds(4 * NP, NP)], t2, sem),
        pltpu.async_copy(tab_ref.at[pl.ds(5 * NP, NP)], t3, sem),
        pltpu.async_copy(tab_ref.at[pl.ds(6 * NP, NP)], t4, sem),
        pltpu.async_copy(tab_ref.at[pl.ds(7 * NP, NP)], t5, sem),
        pltpu.async_copy(perm_ref.at[pl.ds(nbase, NPT)], permb, sem),
    ]

    @pl.when(ew == 0)
    def _():
        pltpu.sync_copy(init0_ref, acc)

    @pl.when(ew != 0)
    def _():
        zv = jnp.zeros((L,), jnp.float32)

        @plsc.parallel_loop(0, NR, 1, unroll=2)
        def _(i):
            for jj in range(128 // L):
                acc[i, pl.ds(jj * L, L)] = zv

    iota16 = lax.iota(jnp.int32, L)
    for g in range(NR // L):
        idxb[pl.ds(g * L, L)] = iota16 + L * g

    for cp in cps:
        cp.wait()

    def _eb(ch):
        return pl.multiple_of(ew * EPW + ch * C, 8)

    def _issue_loads(ch, par):
        eb = _eb(ch)
        sb, db, nb, _ = bufs[par]
        return [
            pltpu.async_copy(src_ref.at[pl.ds(eb, C)], sb, sem3),
            pltpu.async_copy(dst_ref.at[pl.ds(eb, C)], db, sem3),
            pltpu.async_copy(nzE_ref.at[pl.ds(eb, C)], nb, sem3),
        ]

    bufs = [(srcb, dstb, nzb, attb), (srcb1, dstb1, nzb1, attb1)]
    pend = _issue_loads(0, 0)

    @plsc.parallel_loop(0, NPT // L, 1, unroll=2)
    def _(j):
        p = permb[pl.ds(j * L, L)]
        g0 = plsc.load_gather(t2, [p])
        g1 = plsc.load_gather(t3, [p])
        o = nbase + j * L
        t0[pl.ds(o, L)] = t0[pl.ds(o, L)] + g0
        t1[pl.ds(o, L)] = t1[pl.ds(o, L)] + g1
    pltpu.sync_copy(t0.at[pl.ds(nbase, NPT)], sh_up0.at[pl.ds(nbase, NPT)])
    pltpu.sync_copy(t1.at[pl.ds(nbase, NPT)], sh_up1.at[pl.ds(nbase, NPT)])
    pltpu.sync_copy(tab_ref.at[pl.ds(0, NP)], t2)
    pltpu.sync_copy(tab_ref.at[pl.ds(NP, NP)], t3)
    plsc.subcore_barrier()
    dl0 = pltpu.async_copy(sh_up0, t0, sem)
    dl1 = pltpu.async_copy(sh_up1, t1, sem)
    dl0.wait()
    dl1.wait()

    nch = EPW // C
    st = []
    for ch in range(nch):
        par = ch & 1
        for cp in pend:
            cp.wait()
        if ch + 1 < nch:
            pend = _issue_loads(ch + 1, 1 - par)
        for cp in st:
            cp.wait()
        st = []
        sb, db, nb, ab = bufs[par]

        @plsc.parallel_loop(0, C // L, 1, unroll=8)
        def _(j):
            s = sb[pl.ds(j * L, L)]
            d = db[pl.ds(j * L, L)]
            esv = plsc.load_gather(t4, [s])
            edv = plsc.load_gather(t5, [d])
            lg = esv + edv + nb[pl.ds(j * L, L)]
            att = 1.0 / (1.0 + jnp.exp(-lg))
            ab[pl.ds(j * L, L)] = att
            uc0 = plsc.load_gather(t2, [s])
            uc1 = plsc.load_gather(t3, [s])
            up0 = plsc.load_gather(t0, [s])
            up1 = plsc.load_gather(t1, [s])
            r = lax.shift_right_logical(d, 5)
            lb = lax.shift_left(d & 31, 2)
            plsc.addupdate_scatter(acc, [r, lb], att * uc0)
            plsc.addupdate_scatter(acc, [r, lb + 1], att * uc1)
            plsc.addupdate_scatter(acc, [r, lb + 2], att * up0)
            plsc.addupdate_scatter(acc, [r, lb + 3], att * up1)

        st = [pltpu.async_copy(ab, att_out.at[pl.ds(_eb(ch), C)], sem2)]
    for cp in st:
        cp.wait()

    @pl.when(wid == 0)
    def _():
        pltpu.sync_copy(acc, sh_acc)

    plsc.subcore_barrier()

    @pl.when(wid != 0)
    def _():
        pltpu.sync_copy(acc, sh_acc.at[idxb], add=True)

    plsc.subcore_barrier()

    @pl.when(wid < 8)
    def _():
        ob = pl.multiple_of(wid * 40, 8)
        pltpu.sync_copy(sh_acc.at[pl.ds(ob, 40)], tmpacc)
        pltpu.sync_copy(tmpacc, acc_out.at[cid, pl.ds(ob, 40)])


def _sc_edges(table_t, perm_pad, src, dst, nz, init0):
    mesh = plsc.VectorSubcoreMesh(core_axis_name="c", subcore_axis_name="s",
                                  num_cores=2)
    f32 = jnp.float32
    i32 = jnp.int32
    kfn = pl.kernel(
        _sc_body,
        compiler_params=pltpu.CompilerParams(needs_layout_passes=False),
        out_type=[
            jax.ShapeDtypeStruct((2, NR, 128), f32),
            jax.ShapeDtypeStruct((E,), f32),
        ],
        mesh=mesh,
        scratch_types=[
            pltpu.VMEM((NP,), f32),
            pltpu.VMEM((NP,), f32),
            pltpu.VMEM((NP,), f32),
            pltpu.VMEM((NP,), f32),
            pltpu.VMEM((NP,), f32),
            pltpu.VMEM((NP,), f32),
            pltpu.VMEM((NR, 128), f32),
            pltpu.VMEM((C,), i32),
            pltpu.VMEM((C,), i32),
            pltpu.VMEM((C,), f32),
            pltpu.VMEM((C,), f32),
            pltpu.VMEM((C,), i32),
            pltpu.VMEM((C,), i32),
            pltpu.VMEM((C,), f32),
            pltpu.VMEM((C,), f32),
            pltpu.VMEM((NPT,), i32),
            pltpu.VMEM((40, 128), f32),
            pltpu.VMEM((NR,), i32),
            pltpu.SemaphoreType.DMA,
            pltpu.SemaphoreType.DMA,
            pltpu.SemaphoreType.DMA,
            pltpu.VMEM_SHARED((NP,), f32),
            pltpu.VMEM_SHARED((NP,), f32),
            pltpu.VMEM_SHARED((NR, 128), f32),
        ],
    )
    return kfn(table_t, perm_pad, src, dst, nz, init0)


def _tc_merge_body(p_ref, out_ref):
    out_ref[...] = p_ref[0] + p_ref[1]


def _tc_merge(parts):
    return pl.pallas_call(
        _tc_merge_body,
        out_shape=jax.ShapeDtypeStruct((NR, 128), jnp.float32),
    )(parts)


def kernel(x, edge_index, tg_mask, batch, ptr,
           feature_fc_w, feature_fc_b, edge_fc_w, edge_fc_b,
           gat_w, gat_b, gat_out_w, gat_out_b,
           inv_w, inv_b, mix_w, mix_b):
    n0_pad, n1, perm_pad = _get_consts()
    f32 = jnp.float32

    feature_mask, tab_t, init0, e_src, e_dst = _tc_dense(
        x, jnp.asarray(n0_pad), feature_fc_w, feature_fc_b, gat_w, gat_b,
        gat_out_w, gat_out_b, edge_fc_w, edge_fc_b, inv_w, inv_b, mix_w, mix_b,
        edge_index)
    table_t = tab_t.reshape(-1)
    parts, att = _sc_edges(table_t, jnp.asarray(perm_pad),
                           e_src, e_dst, jnp.asarray(n1), init0)
    acc2d = _tc_merge(parts)
    accr = acc2d.reshape(NP, 4)[:N]
    perm_pred = accr[:, 2:4]
    xc_pred = accr[:, 0:2]
    edge_att = att.reshape(E, 1)
    return (perm_pred, xc_pred, feature_mask, edge_att)
